# baseline (speedup 1.0000x reference)
import functools

import jax
import jax.numpy as jnp
from jax import lax
from jax.experimental import pallas as pl
from jax.experimental.pallas import tpu as pltpu
from jax.experimental.pallas import tpu_sc as plsc

N = 10000
E = 320000
H = 128
NC = 2
NS = 16
NW = NC * NS
C = 128
NCHUNK = E // C
BASE_CHUNKS = NCHUNK // NW
EXTRA = NCHUNK - BASE_CHUNKS * NW
NP = 10240
RPT = NP // NS

R = 1000
GRID = N // R


def _mesh():
    return plsc.VectorSubcoreMesh(
        core_axis_name="c", subcore_axis_name="s", num_cores=NC, num_subcores=NS
    )


EP = 327680
CPW = EP // C // NW
KB = 2
GROUPS = CPW // KB


@functools.lru_cache(maxsize=None)
def _seg_sum_kernel():
    @functools.partial(
        pl.kernel,
        out_type=jax.ShapeDtypeStruct((2 * NP, H), jnp.float32),
        mesh=_mesh(),
        scratch_types=[
            pltpu.VMEM((C,), jnp.int32),
            pltpu.VMEM((C,), jnp.int32),
            pltpu.VMEM((C, H), jnp.float32),
            pltpu.VMEM_SHARED((NP, H), jnp.float32),
            pltpu.SemaphoreType.DMA,
        ],
        name="sc_seg_sum",
    )
    def k(src_hbm, dst_hbm, h_hbm, zeros_hbm, out_hbm,
          src_v, dst_v, rows_v, part_s, sem_g):
        cid = lax.axis_index("c")
        sid = lax.axis_index("s")
        w = sid * NC + cid

        pltpu.sync_copy(zeros_hbm, part_s.at[pl.ds(sid * RPT, RPT)])
        plsc.subcore_barrier()

        nchunks = CPW + (w < 0).astype(jnp.int32)

        def body(i, carry):
            base = (w + i * NW) * C
            pltpu.sync_copy(src_hbm.at[pl.ds(base, C)], src_v)
            pltpu.sync_copy(dst_hbm.at[pl.ds(base, C)], dst_v)
            pltpu.async_copy(h_hbm.at[src_v], rows_v, sem_g).wait()
            pltpu.sync_copy(rows_v, part_s.at[dst_v], add=True)
            return carry

        lax.fori_loop(0, nchunks, body, 0)
        plsc.subcore_barrier()
        row0 = cid * NP + sid * RPT
        pltpu.sync_copy(part_s.at[pl.ds(sid * RPT, RPT)],
                        out_hbm.at[pl.ds(row0, RPT)])

    return k


@functools.lru_cache(maxsize=None)
def _feas_kernel():
    @functools.partial(
        pl.kernel,
        out_type=jax.ShapeDtypeStruct((NW * 16,), jnp.float32),
        mesh=_mesh(),
        scratch_types=[
            pltpu.VMEM((C,), jnp.int32),
            pltpu.VMEM((C,), jnp.int32),
            pltpu.VMEM((C,), jnp.float32),
            pltpu.VMEM((C,), jnp.float32),
            pltpu.VMEM((16,), jnp.float32),
            pltpu.SemaphoreType.DMA,
        ],
        name="sc_feas",
    )
    def k(probs_hbm, src_hbm, dst_hbm, out_hbm,
          si_v, di_v, sv_v, dv_v, acc_v, sem):
        cid = lax.axis_index("c")
        sid = lax.axis_index("s")
        w = sid * NC + cid
        nchunks = BASE_CHUNKS + (w < EXTRA).astype(jnp.int32)

        def body(i, acc):
            base = (w + i * NW) * C
            pltpu.sync_copy(src_hbm.at[pl.ds(base, C)], si_v)
            pltpu.sync_copy(dst_hbm.at[pl.ds(base, C)], di_v)
            pltpu.async_copy(probs_hbm.at[si_v], sv_v, sem).wait()
            pltpu.async_copy(probs_hbm.at[di_v], dv_v, sem).wait()
            for j in range(C // 16):
                acc = acc + sv_v[pl.ds(j * 16, 16)] * dv_v[pl.ds(j * 16, 16)]
            return acc

        acc = lax.fori_loop(0, nchunks, body, jnp.zeros((16,), jnp.float32))
        acc_v[...] = acc
        pltpu.sync_copy(acc_v, out_hbm.at[pl.ds(w * 16, 16)])

    return k


def _ln(t, g, b, eps=1e-5):
    m = jnp.mean(t, axis=-1, keepdims=True)
    tc = t - m
    v = jnp.mean(tc * tc, axis=-1, keepdims=True)
    return tc * lax.rsqrt(v + eps) * g + b


def _full(shape):
    return pl.BlockSpec(shape, lambda i: (0,) * len(shape))


def _rows(width):
    return pl.BlockSpec((R, width), lambda i: (i, 0))


def _dot(a, b):
    return jnp.dot(a, b, preferred_element_type=jnp.float32)


def _embed_body(x_ref, xw_ref, xb_ref, g_ref, b_ref, wpx_ref, c0_ref):
    xx = x_ref[...]
    e = xx[:, 0:1] * xw_ref[0:1, :] + xx[:, 1:2] * xw_ref[1:2, :] + xb_ref[...]
    e = _ln(e, g_ref[...], b_ref[...])
    c0_ref[...] = _dot(e, wpx_ref[...])


def _embed(x, xw, xb, g, b, wpx):
    return pl.pallas_call(
        _embed_body,
        grid=(GRID,),
        in_specs=[_rows(2), _full((2, H)), _full((1, H)), _full((1, H)),
                  _full((1, H)), _full((H, H))],
        out_specs=_rows(H),
        out_shape=jax.ShapeDtypeStruct((N, H), jnp.float32),
    )(x, xw, xb, g, b, wpx)


def _stepin_body(c0_ref, y_ref, z_ref, wpy_ref, wpz_ref, bp_ref, g_ref, b_ref,
                 h_ref):
    t = (c0_ref[...] + jax.nn.sigmoid(y_ref[...]) * wpy_ref[...]
         + _dot(z_ref[...], wpz_ref[...]) + bp_ref[...])
    h_ref[...] = _ln(t, g_ref[...], b_ref[...])


def _stepin(c0, y, z, wpy, wpz, bp, g, b):
    return pl.pallas_call(
        _stepin_body,
        grid=(GRID,),
        in_specs=[_rows(H), _rows(1), _rows(H), _full((1, H)), _full((H, H)),
                  _full((1, H)), _full((1, H)), _full((1, H))],
        out_specs=_rows(H),
        out_shape=jax.ShapeDtypeStruct((N, H), jnp.float32),
    )(c0, y, z, wpy, wpz, bp, g, b)


def _gin_body(h_ref, p0_ref, p1_ref, eps_ref, w1_ref, b1_ref, g1_ref, bb1_ref,
              w2_ref, b2_ref, pg_ref, pb_ref, out_ref):
    h = h_ref[...]
    u = (1.0 + eps_ref[0, 0]) * h + p0_ref[...] + p1_ref[...]
    t = _dot(u, w1_ref[...]) + b1_ref[...]
    t = jax.nn.gelu(_ln(t, g1_ref[...], bb1_ref[...]))
    v = _dot(t, w2_ref[...]) + b2_ref[...]
    out_ref[...] = _ln(h + jax.nn.gelu(v), pg_ref[...], pb_ref[...])


def _gin_post(h, p0, p1, eps, w1, b1, g1, bb1, w2, b2, pg, pb):
    return pl.pallas_call(
        _gin_body,
        grid=(GRID,),
        in_specs=[_rows(H), _rows(H), _rows(H),
                  pl.BlockSpec(memory_space=pltpu.SMEM),
                  _full((H, 2 * H)), _full((1, 2 * H)), _full((1, 2 * H)),
                  _full((1, 2 * H)), _full((2 * H, H)), _full((1, H)),
                  _full((1, H)), _full((1, H))],
        out_specs=_rows(H),
        out_shape=jax.ShapeDtypeStruct((N, H), jnp.float32),
    )(h, p0, p1, eps, w1, b1, g1, bb1, w2, b2, pg, pb)


def _outstep_body(y_ref, z_ref, woy_ref, woz_ref, bo_ref, og_ref, ob_ref,
                  w1_ref, b1_ref, w2_ref, b2_ref, yo_ref):
    t = (y_ref[...] * woy_ref[...] + _dot(z_ref[...], woz_ref[...])
         + bo_ref[...])
    t = _ln(t, og_ref[...], ob_ref[...])
    g = jax.nn.gelu(_dot(t, w1_ref[...]) + b1_ref[...])
    yo_ref[...] = _dot(g, w2_ref[...]) + b2_ref[0, 0]


def _outstep(y, z, woy, woz, bo, og, ob, w1, b1, w2, b2):
    return pl.pallas_call(
        _outstep_body,
        grid=(GRID,),
        in_specs=[_rows(1), _rows(H), _full((1, H)), _full((H, H)),
                  _full((1, H)), _full((1, H)), _full((1, H)),
                  _full((H, H)), _full((1, H)), _full((H, 1)),
                  pl.BlockSpec(memory_space=pltpu.SMEM)],
        out_specs=_rows(1),
        out_shape=jax.ShapeDtypeStruct((N, 1), jnp.float32),
    )(y, z, woy, woz, bo, og, ob, w1, b1, w2, b2)


def _probs_body(y_ref, p_ref):
    p_ref[...] = jax.nn.sigmoid(jnp.clip(y_ref[...], -10.0, 10.0))


def _probs(y):
    return pl.pallas_call(
        _probs_body,
        grid=(GRID,),
        in_specs=[_rows(1)],
        out_specs=_rows(1),
        out_shape=jax.ShapeDtypeStruct((N, 1), jnp.float32),
    )(y)


def _loss_body(y_ref, lab_ref, fp_ref, a_ref, b_ref, p_ref, f_ref):
    i = pl.program_id(0)
    l = jnp.clip(y_ref[...], -10.0, 10.0)
    lab = lab_ref[...].astype(jnp.float32)
    a = jnp.sum(lab * jax.nn.softplus(-l))
    b = jnp.sum((1.0 - lab) * jax.nn.softplus(l))
    p = jnp.sum(lab)

    @pl.when(i == 0)
    def _():
        zz = jnp.zeros((1, 1), jnp.float32)
        a_ref[...] = zz
        b_ref[...] = zz
        p_ref[...] = zz
        f_ref[...] = jnp.sum(fp_ref[...]).reshape(1, 1)

    a_ref[...] = a_ref[...] + a
    b_ref[...] = b_ref[...] + b
    p_ref[...] = p_ref[...] + p


def _loss(y, labels2d, feas_parts):
    s = jax.ShapeDtypeStruct((1, 1), jnp.float32)
    one = pl.BlockSpec((1, 1), lambda i: (0, 0))
    return pl.pallas_call(
        _loss_body,
        grid=(GRID,),
        in_specs=[_rows(1), _rows(1), pl.BlockSpec((1, NW * 16), lambda i: (0, 0))],
        out_specs=(one, one, one, one),
        out_shape=(s, s, s, s),
    )(y, labels2d, feas_parts)


def kernel(x, edge_index, y_carry, z_carry, labels, H_step, params):
    p = params
    src = edge_index[0]
    dst = edge_index[1]
    zeros = jnp.zeros((RPT, H), jnp.float32)
    pad = EP - E
    pad_dst = N + (jnp.arange(pad, dtype=jnp.int32) % (NP - N))
    srcp = jnp.concatenate([src, jnp.zeros((pad,), jnp.int32)])
    dstp = jnp.concatenate([dst, pad_dst])

    wp = p["latent_proj_w"]
    wpx, wpy, wpz = wp[:H], wp[H:H + 1], wp[H + 1:]
    bp = p["latent_proj_b"].reshape(1, H)
    lng, lnb = p["latent_norm_g"].reshape(1, H), p["latent_norm_b"].reshape(1, H)

    wo = p["output_proj_w"]
    woy, woz = wo[:1], wo[1:]
    bo = p["output_norm_b"]

    c0 = _embed(x, p["x_embed_w"], p["x_embed_b"].reshape(1, H),
                p["x_norm_g"].reshape(1, H), p["x_norm_b"].reshape(1, H), wpx)

    seg = _seg_sum_kernel()
    feask = _feas_kernel()

    gins = []
    for gp in p["gin"]:
        gins.append((
            gp["eps"].reshape(1, 1),
            gp["w1"], gp["b1"].reshape(1, 2 * H),
            gp["ln_g"].reshape(1, 2 * H), gp["ln_b"].reshape(1, 2 * H),
            gp["w2"], gp["b2"].reshape(1, H),
            gp["post_ln_g"].reshape(1, H), gp["post_ln_b"].reshape(1, H),
        ))

    y, z = y_carry, z_carry
    L_CYCLES, H_CYCLES = 6, 3
    for _ in range(H_CYCLES):
        for _ in range(L_CYCLES):
            h = _stepin(c0, y, z, wpy, wpz, bp, lng, lnb)
            for (eps, w1, b1, g1, bb1, w2, b2, pg, pb) in gins:
                parts = seg(srcp, dstp, h, zeros)
                h = _gin_post(h, parts[:N], parts[NP:NP + N], eps,
                              w1, b1, g1, bb1, w2, b2, pg, pb)
            z = h
        y = _outstep(y, z, woy, woz, p["output_proj_b"].reshape(1, H),
                     p["output_norm_g"].reshape(1, H),
                     p["output_norm_b"].reshape(1, H),
                     p["head_w1"], p["head_b1"].reshape(1, H),
                     p["head_w2"], p["head_b2"].reshape(1, 1))

    probs = _probs(y)
    feas_parts = feask(probs.reshape(-1), src, dst)
    a, b, pcnt, fsum = _loss(y, labels.reshape(N, 1), feas_parts.reshape(1, NW * 16))

    pos = jnp.clip(pcnt[0, 0], 1.0, None)
    neg = jnp.clip(float(N) - pos, 1.0, None)
    pw = neg / pos
    bce = (pw * a[0, 0] + b[0, 0]) / float(N)
    feas = fsum[0, 0] / float(E)
    return bce + 50.0 * feas


# speedup vs baseline: 1.9504x; 1.9504x over previous
import functools

import jax
import jax.numpy as jnp
from jax import lax
from jax.experimental import pallas as pl
from jax.experimental.pallas import tpu as pltpu
from jax.experimental.pallas import tpu_sc as plsc

N = 10000
E = 320000
H = 128
NC = 2
NS = 16
NW = NC * NS
C = 128
NCHUNK = E // C
BASE_CHUNKS = NCHUNK // NW
EXTRA = NCHUNK - BASE_CHUNKS * NW
NP = 10240
RPT = NP // NS

R = 1000
GRID = N // R


def _mesh():
    return plsc.VectorSubcoreMesh(
        core_axis_name="c", subcore_axis_name="s", num_cores=NC, num_subcores=NS
    )


EP = 327680
CPW = EP // C // NW
KB = 2
GROUPS = CPW // KB


@functools.lru_cache(maxsize=None)
def _seg_sum_kernel():
    @functools.partial(
        pl.kernel,
        out_type=jax.ShapeDtypeStruct((2 * NP, H), jnp.float32),
        mesh=_mesh(),
        scratch_types=[
            pltpu.VMEM((C,), jnp.int32),
            pltpu.VMEM((C,), jnp.int32),
            pltpu.VMEM((C, H), jnp.float32),
            pltpu.VMEM_SHARED((NP, H), jnp.float32),
            pltpu.SemaphoreType.DMA,
        ],
        name="sc_seg_sum",
    )
    def k(src_hbm, dst_hbm, h_hbm, zeros_hbm, out_hbm,
          src_v, dst_v, rows_v, part_s, sem_g):
        cid = lax.axis_index("c")
        sid = lax.axis_index("s")
        w = sid * NC + cid

        pltpu.sync_copy(zeros_hbm, part_s.at[pl.ds(sid * RPT, RPT)])
        plsc.subcore_barrier()

        nchunks = CPW + (w < 0).astype(jnp.int32)

        def body(i, carry):
            base = (w + i * NW) * C
            pltpu.sync_copy(src_hbm.at[pl.ds(base, C)], src_v)
            pltpu.sync_copy(dst_hbm.at[pl.ds(base, C)], dst_v)
            pltpu.async_copy(h_hbm.at[src_v], rows_v, sem_g).wait()
            pltpu.sync_copy(rows_v, part_s.at[dst_v], add=True)
            return carry

        lax.fori_loop(0, nchunks, body, 0)
        plsc.subcore_barrier()
        row0 = cid * NP + sid * RPT
        pltpu.sync_copy(part_s.at[pl.ds(sid * RPT, RPT)],
                        out_hbm.at[pl.ds(row0, RPT)])

    return k


@functools.lru_cache(maxsize=None)
def _feas_kernel():
    @functools.partial(
        pl.kernel,
        out_type=jax.ShapeDtypeStruct((NW * 16,), jnp.float32),
        mesh=_mesh(),
        scratch_types=[
            pltpu.VMEM((C,), jnp.int32),
            pltpu.VMEM((C,), jnp.int32),
            pltpu.VMEM((C,), jnp.float32),
            pltpu.VMEM((C,), jnp.float32),
            pltpu.VMEM((16,), jnp.float32),
            pltpu.SemaphoreType.DMA,
        ],
        name="sc_feas",
    )
    def k(probs_hbm, src_hbm, dst_hbm, out_hbm,
          si_v, di_v, sv_v, dv_v, acc_v, sem):
        cid = lax.axis_index("c")
        sid = lax.axis_index("s")
        w = sid * NC + cid
        nchunks = BASE_CHUNKS + (w < EXTRA).astype(jnp.int32)

        def body(i, acc):
            base = (w + i * NW) * C
            pltpu.sync_copy(src_hbm.at[pl.ds(base, C)], si_v)
            pltpu.sync_copy(dst_hbm.at[pl.ds(base, C)], di_v)
            pltpu.async_copy(probs_hbm.at[si_v], sv_v, sem).wait()
            pltpu.async_copy(probs_hbm.at[di_v], dv_v, sem).wait()
            for j in range(C // 16):
                acc = acc + sv_v[pl.ds(j * 16, 16)] * dv_v[pl.ds(j * 16, 16)]
            return acc

        acc = lax.fori_loop(0, nchunks, body, jnp.zeros((16,), jnp.float32))
        acc_v[...] = acc
        pltpu.sync_copy(acc_v, out_hbm.at[pl.ds(w * 16, 16)])

    return k


def _ln(t, g, b, eps=1e-5):
    m = jnp.mean(t, axis=-1, keepdims=True)
    tc = t - m
    v = jnp.mean(tc * tc, axis=-1, keepdims=True)
    return tc * lax.rsqrt(v + eps) * g + b


def _full(shape):
    return pl.BlockSpec(shape, lambda i: (0,) * len(shape))


def _rows(width):
    return pl.BlockSpec((R, width), lambda i: (i, 0))


def _dot(a, b):
    return jnp.dot(a, b, preferred_element_type=jnp.float32)


def _embed_body(x_ref, xw_ref, xb_ref, g_ref, b_ref, wpx_ref, c0_ref):
    xx = x_ref[...]
    e = xx[:, 0:1] * xw_ref[0:1, :] + xx[:, 1:2] * xw_ref[1:2, :] + xb_ref[...]
    e = _ln(e, g_ref[...], b_ref[...])
    c0_ref[...] = _dot(e, wpx_ref[...])


def _embed(x, xw, xb, g, b, wpx):
    return pl.pallas_call(
        _embed_body,
        grid=(GRID,),
        in_specs=[_rows(2), _full((2, H)), _full((1, H)), _full((1, H)),
                  _full((1, H)), _full((H, H))],
        out_specs=_rows(H),
        out_shape=jax.ShapeDtypeStruct((N, H), jnp.float32),
    )(x, xw, xb, g, b, wpx)


def _stepin_body(c0_ref, y_ref, z_ref, wpy_ref, wpz_ref, bp_ref, g_ref, b_ref,
                 h_ref):
    t = (c0_ref[...] + jax.nn.sigmoid(y_ref[...]) * wpy_ref[...]
         + _dot(z_ref[...], wpz_ref[...]) + bp_ref[...])
    h_ref[...] = _ln(t, g_ref[...], b_ref[...])


def _stepin(c0, y, z, wpy, wpz, bp, g, b):
    return pl.pallas_call(
        _stepin_body,
        grid=(GRID,),
        in_specs=[_rows(H), _rows(1), _rows(H), _full((1, H)), _full((H, H)),
                  _full((1, H)), _full((1, H)), _full((1, H))],
        out_specs=_rows(H),
        out_shape=jax.ShapeDtypeStruct((N, H), jnp.float32),
    )(c0, y, z, wpy, wpz, bp, g, b)


def _gin_body(h_ref, p0_ref, p1_ref, eps_ref, w1_ref, b1_ref, g1_ref, bb1_ref,
              w2_ref, b2_ref, pg_ref, pb_ref, out_ref):
    h = h_ref[...]
    u = (1.0 + eps_ref[0, 0]) * h + p0_ref[...] + p1_ref[...]
    t = _dot(u, w1_ref[...]) + b1_ref[...]
    t = jax.nn.gelu(_ln(t, g1_ref[...], bb1_ref[...]))
    v = _dot(t, w2_ref[...]) + b2_ref[...]
    out_ref[...] = _ln(h + jax.nn.gelu(v), pg_ref[...], pb_ref[...])


def _gin_post(h, p0, p1, eps, w1, b1, g1, bb1, w2, b2, pg, pb):
    return pl.pallas_call(
        _gin_body,
        grid=(GRID,),
        in_specs=[_rows(H), _rows(H), _rows(H),
                  pl.BlockSpec(memory_space=pltpu.SMEM),
                  _full((H, 2 * H)), _full((1, 2 * H)), _full((1, 2 * H)),
                  _full((1, 2 * H)), _full((2 * H, H)), _full((1, H)),
                  _full((1, H)), _full((1, H))],
        out_specs=_rows(H),
        out_shape=jax.ShapeDtypeStruct((N, H), jnp.float32),
    )(h, p0, p1, eps, w1, b1, g1, bb1, w2, b2, pg, pb)


def _outstep_body(y_ref, z_ref, woy_ref, woz_ref, bo_ref, og_ref, ob_ref,
                  w1_ref, b1_ref, w2_ref, b2_ref, yo_ref):
    t = (y_ref[...] * woy_ref[...] + _dot(z_ref[...], woz_ref[...])
         + bo_ref[...])
    t = _ln(t, og_ref[...], ob_ref[...])
    g = jax.nn.gelu(_dot(t, w1_ref[...]) + b1_ref[...])
    yo_ref[...] = _dot(g, w2_ref[...]) + b2_ref[0, 0]


def _outstep(y, z, woy, woz, bo, og, ob, w1, b1, w2, b2):
    return pl.pallas_call(
        _outstep_body,
        grid=(GRID,),
        in_specs=[_rows(1), _rows(H), _full((1, H)), _full((H, H)),
                  _full((1, H)), _full((1, H)), _full((1, H)),
                  _full((H, H)), _full((1, H)), _full((H, 1)),
                  pl.BlockSpec(memory_space=pltpu.SMEM)],
        out_specs=_rows(1),
        out_shape=jax.ShapeDtypeStruct((N, 1), jnp.float32),
    )(y, z, woy, woz, bo, og, ob, w1, b1, w2, b2)


def _probs_body(y_ref, p_ref):
    p_ref[...] = jax.nn.sigmoid(jnp.clip(y_ref[...], -10.0, 10.0))


def _probs(y):
    return pl.pallas_call(
        _probs_body,
        grid=(GRID,),
        in_specs=[_rows(1)],
        out_specs=_rows(1),
        out_shape=jax.ShapeDtypeStruct((N, 1), jnp.float32),
    )(y)


def _loss_body(y_ref, lab_ref, fp_ref, a_ref, b_ref, p_ref, f_ref):
    i = pl.program_id(0)
    l = jnp.clip(y_ref[...], -10.0, 10.0)
    lab = lab_ref[...].astype(jnp.float32)
    a = jnp.sum(lab * jax.nn.softplus(-l))
    b = jnp.sum((1.0 - lab) * jax.nn.softplus(l))
    p = jnp.sum(lab)

    @pl.when(i == 0)
    def _():
        zz = jnp.zeros((1, 1), jnp.float32)
        a_ref[...] = zz
        b_ref[...] = zz
        p_ref[...] = zz
        f_ref[...] = jnp.sum(fp_ref[...]).reshape(1, 1)

    a_ref[...] = a_ref[...] + a
    b_ref[...] = b_ref[...] + b
    p_ref[...] = p_ref[...] + p


def _loss(y, labels2d, feas_parts):
    s = jax.ShapeDtypeStruct((1, 1), jnp.float32)
    one = pl.BlockSpec((1, 1), lambda i: (0, 0))
    return pl.pallas_call(
        _loss_body,
        grid=(GRID,),
        in_specs=[_rows(1), _rows(1), pl.BlockSpec((1, NW * 16), lambda i: (0, 0))],
        out_specs=(one, one, one, one),
        out_shape=(s, s, s, s),
    )(y, labels2d, feas_parts)


def kernel(x, edge_index, y_carry, z_carry, labels, H_step, params):
    p = params
    src = edge_index[0]
    dst = edge_index[1]
    zeros = jnp.zeros((RPT, H), jnp.float32)
    pad = EP - E
    pad_dst = N + (jnp.arange(pad, dtype=jnp.int32) % (NP - N))
    pad_src = jnp.arange(pad, dtype=jnp.int32) % N
    srcp = jnp.concatenate([src, pad_src])
    dstp = jnp.concatenate([dst, pad_dst])

    wp = p["latent_proj_w"]
    wpx, wpy, wpz = wp[:H], wp[H:H + 1], wp[H + 1:]
    bp = p["latent_proj_b"].reshape(1, H)
    lng, lnb = p["latent_norm_g"].reshape(1, H), p["latent_norm_b"].reshape(1, H)

    wo = p["output_proj_w"]
    woy, woz = wo[:1], wo[1:]
    bo = p["output_norm_b"]

    c0 = _embed(x, p["x_embed_w"], p["x_embed_b"].reshape(1, H),
                p["x_norm_g"].reshape(1, H), p["x_norm_b"].reshape(1, H), wpx)

    seg = _seg_sum_kernel()
    feask = _feas_kernel()

    gins = []
    for gp in p["gin"]:
        gins.append((
            gp["eps"].reshape(1, 1),
            gp["w1"], gp["b1"].reshape(1, 2 * H),
            gp["ln_g"].reshape(1, 2 * H), gp["ln_b"].reshape(1, 2 * H),
            gp["w2"], gp["b2"].reshape(1, H),
            gp["post_ln_g"].reshape(1, H), gp["post_ln_b"].reshape(1, H),
        ))

    y, z = y_carry, z_carry
    L_CYCLES, H_CYCLES = 6, 3
    for _ in range(H_CYCLES):
        for _ in range(L_CYCLES):
            h = _stepin(c0, y, z, wpy, wpz, bp, lng, lnb)
            for (eps, w1, b1, g1, bb1, w2, b2, pg, pb) in gins:
                parts = seg(srcp, dstp, h, zeros)
                h = _gin_post(h, parts[:N], parts[NP:NP + N], eps,
                              w1, b1, g1, bb1, w2, b2, pg, pb)
            z = h
        y = _outstep(y, z, woy, woz, p["output_proj_b"].reshape(1, H),
                     p["output_norm_g"].reshape(1, H),
                     p["output_norm_b"].reshape(1, H),
                     p["head_w1"], p["head_b1"].reshape(1, H),
                     p["head_w2"], p["head_b2"].reshape(1, 1))

    probs = _probs(y)
    feas_parts = feask(probs.reshape(-1), src, dst)
    a, b, pcnt, fsum = _loss(y, labels.reshape(N, 1), feas_parts.reshape(1, NW * 16))

    pos = jnp.clip(pcnt[0, 0], 1.0, None)
    neg = jnp.clip(float(N) - pos, 1.0, None)
    pw = neg / pos
    bce = (pw * a[0, 0] + b[0, 0]) / float(N)
    feas = fsum[0, 0] / float(E)
    return bce + 50.0 * feas


# speedup vs baseline: 2.8099x; 1.4407x over previous
import functools

import jax
import jax.numpy as jnp
from jax import lax
from jax.experimental import pallas as pl
from jax.experimental.pallas import tpu as pltpu
from jax.experimental.pallas import tpu_sc as plsc

N = 10000
E = 320000
H = 128
NC = 2
NS = 16
NW = NC * NS
C = 128
NCHUNK = E // C
BASE_CHUNKS = NCHUNK // NW
EXTRA = NCHUNK - BASE_CHUNKS * NW
NP = 10240
RPT = NP // NS

R = 1000
GRID = N // R


def _mesh():
    return plsc.VectorSubcoreMesh(
        core_axis_name="c", subcore_axis_name="s", num_cores=NC, num_subcores=NS
    )


EP = 327680
CPW = EP // C // NW
KB = 2
GROUPS = CPW // KB


@functools.lru_cache(maxsize=None)
def _seg_sum_kernel():
    @functools.partial(
        pl.kernel,
        out_type=jax.ShapeDtypeStruct((2 * NP, H), jnp.float32),
        mesh=_mesh(),
        scratch_types=[
            pltpu.VMEM((C,), jnp.int32),
            pltpu.VMEM((C,), jnp.int32),
            pltpu.VMEM((C, H), jnp.float32),
            pltpu.VMEM((C,), jnp.int32),
            pltpu.VMEM((C,), jnp.int32),
            pltpu.VMEM((C, H), jnp.float32),
            pltpu.VMEM_SHARED((NP, H), jnp.float32),
            pltpu.SemaphoreType.DMA,
            pltpu.SemaphoreType.DMA,
        ],
        name="sc_seg_sum",
    )
    def k(src_hbm, dst_hbm, h_hbm, zeros_hbm, out_hbm,
          src0_v, dst0_v, rows0_v, src1_v, dst1_v, rows1_v,
          part_s, sem_g0, sem_g1):
        cid = lax.axis_index("c")
        sid = lax.axis_index("s")
        w = sid * NC + cid

        pltpu.sync_copy(zeros_hbm, part_s.at[pl.ds(sid * RPT, RPT)])
        plsc.subcore_barrier()

        ngroups = (CPW // 2) + (w < 0).astype(jnp.int32)

        def body(t, carry):
            b0 = (w + (2 * t) * NW) * C
            b1 = (w + (2 * t + 1) * NW) * C
            pltpu.sync_copy(src_hbm.at[pl.ds(b0, C)], src0_v)
            g0 = pltpu.async_copy(h_hbm.at[src0_v], rows0_v, sem_g0)
            pltpu.sync_copy(src_hbm.at[pl.ds(b1, C)], src1_v)
            g1 = pltpu.async_copy(h_hbm.at[src1_v], rows1_v, sem_g1)
            pltpu.sync_copy(dst_hbm.at[pl.ds(b0, C)], dst0_v)
            pltpu.sync_copy(dst_hbm.at[pl.ds(b1, C)], dst1_v)
            g0.wait()
            pltpu.sync_copy(rows0_v, part_s.at[dst0_v], add=True)
            g1.wait()
            pltpu.sync_copy(rows1_v, part_s.at[dst1_v], add=True)
            return carry

        lax.fori_loop(0, ngroups, body, 0)
        plsc.subcore_barrier()
        row0 = cid * NP + sid * RPT
        pltpu.sync_copy(part_s.at[pl.ds(sid * RPT, RPT)],
                        out_hbm.at[pl.ds(row0, RPT)])

    return k


@functools.lru_cache(maxsize=None)
def _feas_kernel():
    @functools.partial(
        pl.kernel,
        out_type=jax.ShapeDtypeStruct((NW * 16,), jnp.float32),
        mesh=_mesh(),
        scratch_types=[
            pltpu.VMEM((C,), jnp.int32),
            pltpu.VMEM((C,), jnp.int32),
            pltpu.VMEM((C,), jnp.float32),
            pltpu.VMEM((C,), jnp.float32),
            pltpu.VMEM((16,), jnp.float32),
            pltpu.SemaphoreType.DMA,
        ],
        name="sc_feas",
    )
    def k(probs_hbm, src_hbm, dst_hbm, out_hbm,
          si_v, di_v, sv_v, dv_v, acc_v, sem):
        cid = lax.axis_index("c")
        sid = lax.axis_index("s")
        w = sid * NC + cid
        nchunks = BASE_CHUNKS + (w < EXTRA).astype(jnp.int32)

        def body(i, acc):
            base = (w + i * NW) * C
            pltpu.sync_copy(src_hbm.at[pl.ds(base, C)], si_v)
            pltpu.sync_copy(dst_hbm.at[pl.ds(base, C)], di_v)
            pltpu.async_copy(probs_hbm.at[si_v], sv_v, sem).wait()
            pltpu.async_copy(probs_hbm.at[di_v], dv_v, sem).wait()
            for j in range(C // 16):
                acc = acc + sv_v[pl.ds(j * 16, 16)] * dv_v[pl.ds(j * 16, 16)]
            return acc

        acc = lax.fori_loop(0, nchunks, body, jnp.zeros((16,), jnp.float32))
        acc_v[...] = acc
        pltpu.sync_copy(acc_v, out_hbm.at[pl.ds(w * 16, 16)])

    return k


def _ln(t, g, b, eps=1e-5):
    m = jnp.mean(t, axis=-1, keepdims=True)
    tc = t - m
    v = jnp.mean(tc * tc, axis=-1, keepdims=True)
    return tc * lax.rsqrt(v + eps) * g + b


def _full(shape):
    return pl.BlockSpec(shape, lambda i: (0,) * len(shape))


def _rows(width):
    return pl.BlockSpec((R, width), lambda i: (i, 0))


def _dot(a, b):
    return jnp.dot(a, b, preferred_element_type=jnp.float32)


def _embed_body(x_ref, xw_ref, xb_ref, g_ref, b_ref, wpx_ref, c0_ref):
    xx = x_ref[...]
    e = xx[:, 0:1] * xw_ref[0:1, :] + xx[:, 1:2] * xw_ref[1:2, :] + xb_ref[...]
    e = _ln(e, g_ref[...], b_ref[...])
    c0_ref[...] = _dot(e, wpx_ref[...])


def _embed(x, xw, xb, g, b, wpx):
    return pl.pallas_call(
        _embed_body,
        grid=(GRID,),
        in_specs=[_rows(2), _full((2, H)), _full((1, H)), _full((1, H)),
                  _full((1, H)), _full((H, H))],
        out_specs=_rows(H),
        out_shape=jax.ShapeDtypeStruct((N, H), jnp.float32),
    )(x, xw, xb, g, b, wpx)


def _stepin_body(c0_ref, y_ref, z_ref, wpy_ref, wpz_ref, bp_ref, g_ref, b_ref,
                 h_ref):
    t = (c0_ref[...] + jax.nn.sigmoid(y_ref[...]) * wpy_ref[...]
         + _dot(z_ref[...], wpz_ref[...]) + bp_ref[...])
    h_ref[...] = _ln(t, g_ref[...], b_ref[...])


def _stepin(c0, y, z, wpy, wpz, bp, g, b):
    return pl.pallas_call(
        _stepin_body,
        grid=(GRID,),
        in_specs=[_rows(H), _rows(1), _rows(H), _full((1, H)), _full((H, H)),
                  _full((1, H)), _full((1, H)), _full((1, H))],
        out_specs=_rows(H),
        out_shape=jax.ShapeDtypeStruct((N, H), jnp.float32),
    )(c0, y, z, wpy, wpz, bp, g, b)


def _gin_body(h_ref, p0_ref, p1_ref, eps_ref, w1_ref, b1_ref, g1_ref, bb1_ref,
              w2_ref, b2_ref, pg_ref, pb_ref, out_ref):
    h = h_ref[...]
    u = (1.0 + eps_ref[0, 0]) * h + p0_ref[...] + p1_ref[...]
    t = _dot(u, w1_ref[...]) + b1_ref[...]
    t = jax.nn.gelu(_ln(t, g1_ref[...], bb1_ref[...]))
    v = _dot(t, w2_ref[...]) + b2_ref[...]
    out_ref[...] = _ln(h + jax.nn.gelu(v), pg_ref[...], pb_ref[...])


def _gin_post(h, p0, p1, eps, w1, b1, g1, bb1, w2, b2, pg, pb):
    return pl.pallas_call(
        _gin_body,
        grid=(GRID,),
        in_specs=[_rows(H), _rows(H), _rows(H),
                  pl.BlockSpec(memory_space=pltpu.SMEM),
                  _full((H, 2 * H)), _full((1, 2 * H)), _full((1, 2 * H)),
                  _full((1, 2 * H)), _full((2 * H, H)), _full((1, H)),
                  _full((1, H)), _full((1, H))],
        out_specs=_rows(H),
        out_shape=jax.ShapeDtypeStruct((N, H), jnp.float32),
    )(h, p0, p1, eps, w1, b1, g1, bb1, w2, b2, pg, pb)


def _outstep_body(y_ref, z_ref, woy_ref, woz_ref, bo_ref, og_ref, ob_ref,
                  w1_ref, b1_ref, w2_ref, b2_ref, yo_ref):
    t = (y_ref[...] * woy_ref[...] + _dot(z_ref[...], woz_ref[...])
         + bo_ref[...])
    t = _ln(t, og_ref[...], ob_ref[...])
    g = jax.nn.gelu(_dot(t, w1_ref[...]) + b1_ref[...])
    yo_ref[...] = _dot(g, w2_ref[...]) + b2_ref[0, 0]


def _outstep(y, z, woy, woz, bo, og, ob, w1, b1, w2, b2):
    return pl.pallas_call(
        _outstep_body,
        grid=(GRID,),
        in_specs=[_rows(1), _rows(H), _full((1, H)), _full((H, H)),
                  _full((1, H)), _full((1, H)), _full((1, H)),
                  _full((H, H)), _full((1, H)), _full((H, 1)),
                  pl.BlockSpec(memory_space=pltpu.SMEM)],
        out_specs=_rows(1),
        out_shape=jax.ShapeDtypeStruct((N, 1), jnp.float32),
    )(y, z, woy, woz, bo, og, ob, w1, b1, w2, b2)


def _probs_body(y_ref, p_ref):
    p_ref[...] = jax.nn.sigmoid(jnp.clip(y_ref[...], -10.0, 10.0))


def _probs(y):
    return pl.pallas_call(
        _probs_body,
        grid=(GRID,),
        in_specs=[_rows(1)],
        out_specs=_rows(1),
        out_shape=jax.ShapeDtypeStruct((N, 1), jnp.float32),
    )(y)


def _loss_body(y_ref, lab_ref, fp_ref, a_ref, b_ref, p_ref, f_ref):
    i = pl.program_id(0)
    l = jnp.clip(y_ref[...], -10.0, 10.0)
    lab = lab_ref[...].astype(jnp.float32)
    a = jnp.sum(lab * jax.nn.softplus(-l))
    b = jnp.sum((1.0 - lab) * jax.nn.softplus(l))
    p = jnp.sum(lab)

    @pl.when(i == 0)
    def _():
        zz = jnp.zeros((1, 1), jnp.float32)
        a_ref[...] = zz
        b_ref[...] = zz
        p_ref[...] = zz
        f_ref[...] = jnp.sum(fp_ref[...]).reshape(1, 1)

    a_ref[...] = a_ref[...] + a
    b_ref[...] = b_ref[...] + b
    p_ref[...] = p_ref[...] + p


def _loss(y, labels2d, feas_parts):
    s = jax.ShapeDtypeStruct((1, 1), jnp.float32)
    one = pl.BlockSpec((1, 1), lambda i: (0, 0))
    return pl.pallas_call(
        _loss_body,
        grid=(GRID,),
        in_specs=[_rows(1), _rows(1), pl.BlockSpec((1, NW * 16), lambda i: (0, 0))],
        out_specs=(one, one, one, one),
        out_shape=(s, s, s, s),
    )(y, labels2d, feas_parts)


def kernel(x, edge_index, y_carry, z_carry, labels, H_step, params):
    p = params
    src = edge_index[0]
    dst = edge_index[1]
    zeros = jnp.zeros((RPT, H), jnp.float32)
    pad = EP - E
    pad_dst = N + (jnp.arange(pad, dtype=jnp.int32) % (NP - N))
    pad_src = jnp.arange(pad, dtype=jnp.int32) % N
    srcp = jnp.concatenate([src, pad_src])
    dstp = jnp.concatenate([dst, pad_dst])

    wp = p["latent_proj_w"]
    wpx, wpy, wpz = wp[:H], wp[H:H + 1], wp[H + 1:]
    bp = p["latent_proj_b"].reshape(1, H)
    lng, lnb = p["latent_norm_g"].reshape(1, H), p["latent_norm_b"].reshape(1, H)

    wo = p["output_proj_w"]
    woy, woz = wo[:1], wo[1:]
    bo = p["output_norm_b"]

    c0 = _embed(x, p["x_embed_w"], p["x_embed_b"].reshape(1, H),
                p["x_norm_g"].reshape(1, H), p["x_norm_b"].reshape(1, H), wpx)

    seg = _seg_sum_kernel()
    feask = _feas_kernel()

    gins = []
    for gp in p["gin"]:
        gins.append((
            gp["eps"].reshape(1, 1),
            gp["w1"], gp["b1"].reshape(1, 2 * H),
            gp["ln_g"].reshape(1, 2 * H), gp["ln_b"].reshape(1, 2 * H),
            gp["w2"], gp["b2"].reshape(1, H),
            gp["post_ln_g"].reshape(1, H), gp["post_ln_b"].reshape(1, H),
        ))

    y, z = y_carry, z_carry
    L_CYCLES, H_CYCLES = 6, 3
    for _ in range(H_CYCLES):
        for _ in range(L_CYCLES):
            h = _stepin(c0, y, z, wpy, wpz, bp, lng, lnb)
            for (eps, w1, b1, g1, bb1, w2, b2, pg, pb) in gins:
                parts = seg(srcp, dstp, h, zeros)
                h = _gin_post(h, parts[:N], parts[NP:NP + N], eps,
                              w1, b1, g1, bb1, w2, b2, pg, pb)
            z = h
        y = _outstep(y, z, woy, woz, p["output_proj_b"].reshape(1, H),
                     p["output_norm_g"].reshape(1, H),
                     p["output_norm_b"].reshape(1, H),
                     p["head_w1"], p["head_b1"].reshape(1, H),
                     p["head_w2"], p["head_b2"].reshape(1, 1))

    probs = _probs(y)
    feas_parts = feask(probs.reshape(-1), src, dst)
    a, b, pcnt, fsum = _loss(y, labels.reshape(N, 1), feas_parts.reshape(1, NW * 16))

    pos = jnp.clip(pcnt[0, 0], 1.0, None)
    neg = jnp.clip(float(N) - pos, 1.0, None)
    pw = neg / pos
    bce = (pw * a[0, 0] + b[0, 0]) / float(N)
    feas = fsum[0, 0] / float(E)
    return bce + 50.0 * feas


# speedup vs baseline: 2.8367x; 1.0096x over previous
import functools

import jax
import jax.numpy as jnp
from jax import lax
from jax.experimental import pallas as pl
from jax.experimental.pallas import tpu as pltpu
from jax.experimental.pallas import tpu_sc as plsc

N = 10000
E = 320000
H = 128
NC = 2
NS = 16
NW = NC * NS
C = 128
NCHUNK = E // C
BASE_CHUNKS = NCHUNK // NW
EXTRA = NCHUNK - BASE_CHUNKS * NW
NP = 10240
RPT = NP // NS

R = 1000
GRID = N // R


def _mesh():
    return plsc.VectorSubcoreMesh(
        core_axis_name="c", subcore_axis_name="s", num_cores=NC, num_subcores=NS
    )


EP = 327680
CPW = EP // C // NW
KB = 2
GROUPS = CPW // KB


@functools.lru_cache(maxsize=None)
def _seg_sum_kernel():
    @functools.partial(
        pl.kernel,
        out_type=jax.ShapeDtypeStruct((2 * NP, H), jnp.float32),
        mesh=_mesh(),
        scratch_types=[
            pltpu.VMEM((C,), jnp.int32),
            pltpu.VMEM((C,), jnp.int32),
            pltpu.VMEM((C, H), jnp.float32),
            pltpu.VMEM((C,), jnp.int32),
            pltpu.VMEM((C,), jnp.int32),
            pltpu.VMEM((C, H), jnp.float32),
            pltpu.VMEM_SHARED((NP, H), jnp.float32),
            pltpu.SemaphoreType.DMA,
            pltpu.SemaphoreType.DMA,
            pltpu.SemaphoreType.DMA,
            pltpu.SemaphoreType.DMA,
        ],
        name="sc_seg_sum",
    )
    def k(src_hbm, dst_hbm, h_hbm, zeros_hbm, out_hbm,
          src0_v, dst0_v, rows0_v, src1_v, dst1_v, rows1_v,
          part_s, sem_g0, sem_g1, sem_s0, sem_s1):
        cid = lax.axis_index("c")
        sid = lax.axis_index("s")
        w = sid * NC + cid

        pltpu.sync_copy(zeros_hbm, part_s.at[pl.ds(sid * RPT, RPT)])
        plsc.subcore_barrier()

        ngroups = (CPW // 2) + (w < 0).astype(jnp.int32)

        def body(t, carry):
            b0 = (w + (2 * t) * NW) * C
            b1 = (w + (2 * t + 1) * NW) * C
            pltpu.sync_copy(src_hbm.at[pl.ds(b0, C)], src0_v)
            g0 = pltpu.async_copy(h_hbm.at[src0_v], rows0_v, sem_g0)
            pltpu.sync_copy(src_hbm.at[pl.ds(b1, C)], src1_v)
            g1 = pltpu.async_copy(h_hbm.at[src1_v], rows1_v, sem_g1)
            pltpu.sync_copy(dst_hbm.at[pl.ds(b0, C)], dst0_v)
            pltpu.sync_copy(dst_hbm.at[pl.ds(b1, C)], dst1_v)
            g0.wait()
            s0 = pltpu.async_copy(rows0_v, part_s.at[dst0_v], sem_s0,
                                  add=True)
            g1.wait()
            s1 = pltpu.async_copy(rows1_v, part_s.at[dst1_v], sem_s1,
                                  add=True)
            s0.wait()
            s1.wait()
            return carry

        lax.fori_loop(0, ngroups, body, 0)
        plsc.subcore_barrier()
        row0 = cid * NP + sid * RPT
        pltpu.sync_copy(part_s.at[pl.ds(sid * RPT, RPT)],
                        out_hbm.at[pl.ds(row0, RPT)])

    return k


@functools.lru_cache(maxsize=None)
def _feas_kernel():
    @functools.partial(
        pl.kernel,
        out_type=jax.ShapeDtypeStruct((NW * 16,), jnp.float32),
        mesh=_mesh(),
        scratch_types=[
            pltpu.VMEM((C,), jnp.int32),
            pltpu.VMEM((C,), jnp.int32),
            pltpu.VMEM((C,), jnp.float32),
            pltpu.VMEM((C,), jnp.float32),
            pltpu.VMEM((16,), jnp.float32),
            pltpu.SemaphoreType.DMA,
        ],
        name="sc_feas",
    )
    def k(probs_hbm, src_hbm, dst_hbm, out_hbm,
          si_v, di_v, sv_v, dv_v, acc_v, sem):
        cid = lax.axis_index("c")
        sid = lax.axis_index("s")
        w = sid * NC + cid
        nchunks = BASE_CHUNKS + (w < EXTRA).astype(jnp.int32)

        def body(i, acc):
            base = (w + i * NW) * C
            pltpu.sync_copy(src_hbm.at[pl.ds(base, C)], si_v)
            pltpu.sync_copy(dst_hbm.at[pl.ds(base, C)], di_v)
            pltpu.async_copy(probs_hbm.at[si_v], sv_v, sem).wait()
            pltpu.async_copy(probs_hbm.at[di_v], dv_v, sem).wait()
            for j in range(C // 16):
                acc = acc + sv_v[pl.ds(j * 16, 16)] * dv_v[pl.ds(j * 16, 16)]
            return acc

        acc = lax.fori_loop(0, nchunks, body, jnp.zeros((16,), jnp.float32))
        acc_v[...] = acc
        pltpu.sync_copy(acc_v, out_hbm.at[pl.ds(w * 16, 16)])

    return k


def _ln(t, g, b, eps=1e-5):
    m = jnp.mean(t, axis=-1, keepdims=True)
    tc = t - m
    v = jnp.mean(tc * tc, axis=-1, keepdims=True)
    return tc * lax.rsqrt(v + eps) * g + b


def _full(shape):
    return pl.BlockSpec(shape, lambda i: (0,) * len(shape))


def _rows(width):
    return pl.BlockSpec((R, width), lambda i: (i, 0))


def _dot(a, b):
    return jnp.dot(a, b, preferred_element_type=jnp.float32)


def _embed_body(x_ref, xw_ref, xb_ref, g_ref, b_ref, wpx_ref, c0_ref):
    xx = x_ref[...]
    e = xx[:, 0:1] * xw_ref[0:1, :] + xx[:, 1:2] * xw_ref[1:2, :] + xb_ref[...]
    e = _ln(e, g_ref[...], b_ref[...])
    c0_ref[...] = _dot(e, wpx_ref[...])


def _embed(x, xw, xb, g, b, wpx):
    return pl.pallas_call(
        _embed_body,
        grid=(GRID,),
        in_specs=[_rows(2), _full((2, H)), _full((1, H)), _full((1, H)),
                  _full((1, H)), _full((H, H))],
        out_specs=_rows(H),
        out_shape=jax.ShapeDtypeStruct((N, H), jnp.float32),
    )(x, xw, xb, g, b, wpx)


def _stepin_body(c0_ref, y_ref, z_ref, wpy_ref, wpz_ref, bp_ref, g_ref, b_ref,
                 h_ref):
    t = (c0_ref[...] + jax.nn.sigmoid(y_ref[...]) * wpy_ref[...]
         + _dot(z_ref[...], wpz_ref[...]) + bp_ref[...])
    h_ref[...] = _ln(t, g_ref[...], b_ref[...])


def _stepin(c0, y, z, wpy, wpz, bp, g, b):
    return pl.pallas_call(
        _stepin_body,
        grid=(GRID,),
        in_specs=[_rows(H), _rows(1), _rows(H), _full((1, H)), _full((H, H)),
                  _full((1, H)), _full((1, H)), _full((1, H))],
        out_specs=_rows(H),
        out_shape=jax.ShapeDtypeStruct((N, H), jnp.float32),
    )(c0, y, z, wpy, wpz, bp, g, b)


def _gin_body(h_ref, p0_ref, p1_ref, eps_ref, w1_ref, b1_ref, g1_ref, bb1_ref,
              w2_ref, b2_ref, pg_ref, pb_ref, out_ref):
    h = h_ref[...]
    u = (1.0 + eps_ref[0, 0]) * h + p0_ref[...] + p1_ref[...]
    t = _dot(u, w1_ref[...]) + b1_ref[...]
    t = jax.nn.gelu(_ln(t, g1_ref[...], bb1_ref[...]))
    v = _dot(t, w2_ref[...]) + b2_ref[...]
    out_ref[...] = _ln(h + jax.nn.gelu(v), pg_ref[...], pb_ref[...])


def _gin_post(h, p0, p1, eps, w1, b1, g1, bb1, w2, b2, pg, pb):
    return pl.pallas_call(
        _gin_body,
        grid=(GRID,),
        in_specs=[_rows(H), _rows(H), _rows(H),
                  pl.BlockSpec(memory_space=pltpu.SMEM),
                  _full((H, 2 * H)), _full((1, 2 * H)), _full((1, 2 * H)),
                  _full((1, 2 * H)), _full((2 * H, H)), _full((1, H)),
                  _full((1, H)), _full((1, H))],
        out_specs=_rows(H),
        out_shape=jax.ShapeDtypeStruct((N, H), jnp.float32),
    )(h, p0, p1, eps, w1, b1, g1, bb1, w2, b2, pg, pb)


def _outstep_body(y_ref, z_ref, woy_ref, woz_ref, bo_ref, og_ref, ob_ref,
                  w1_ref, b1_ref, w2_ref, b2_ref, yo_ref):
    t = (y_ref[...] * woy_ref[...] + _dot(z_ref[...], woz_ref[...])
         + bo_ref[...])
    t = _ln(t, og_ref[...], ob_ref[...])
    g = jax.nn.gelu(_dot(t, w1_ref[...]) + b1_ref[...])
    yo_ref[...] = _dot(g, w2_ref[...]) + b2_ref[0, 0]


def _outstep(y, z, woy, woz, bo, og, ob, w1, b1, w2, b2):
    return pl.pallas_call(
        _outstep_body,
        grid=(GRID,),
        in_specs=[_rows(1), _rows(H), _full((1, H)), _full((H, H)),
                  _full((1, H)), _full((1, H)), _full((1, H)),
                  _full((H, H)), _full((1, H)), _full((H, 1)),
                  pl.BlockSpec(memory_space=pltpu.SMEM)],
        out_specs=_rows(1),
        out_shape=jax.ShapeDtypeStruct((N, 1), jnp.float32),
    )(y, z, woy, woz, bo, og, ob, w1, b1, w2, b2)


def _probs_body(y_ref, p_ref):
    p_ref[...] = jax.nn.sigmoid(jnp.clip(y_ref[...], -10.0, 10.0))


def _probs(y):
    return pl.pallas_call(
        _probs_body,
        grid=(GRID,),
        in_specs=[_rows(1)],
        out_specs=_rows(1),
        out_shape=jax.ShapeDtypeStruct((N, 1), jnp.float32),
    )(y)


def _loss_body(y_ref, lab_ref, fp_ref, a_ref, b_ref, p_ref, f_ref):
    i = pl.program_id(0)
    l = jnp.clip(y_ref[...], -10.0, 10.0)
    lab = lab_ref[...].astype(jnp.float32)
    a = jnp.sum(lab * jax.nn.softplus(-l))
    b = jnp.sum((1.0 - lab) * jax.nn.softplus(l))
    p = jnp.sum(lab)

    @pl.when(i == 0)
    def _():
        zz = jnp.zeros((1, 1), jnp.float32)
        a_ref[...] = zz
        b_ref[...] = zz
        p_ref[...] = zz
        f_ref[...] = jnp.sum(fp_ref[...]).reshape(1, 1)

    a_ref[...] = a_ref[...] + a
    b_ref[...] = b_ref[...] + b
    p_ref[...] = p_ref[...] + p


def _loss(y, labels2d, feas_parts):
    s = jax.ShapeDtypeStruct((1, 1), jnp.float32)
    one = pl.BlockSpec((1, 1), lambda i: (0, 0))
    return pl.pallas_call(
        _loss_body,
        grid=(GRID,),
        in_specs=[_rows(1), _rows(1), pl.BlockSpec((1, NW * 16), lambda i: (0, 0))],
        out_specs=(one, one, one, one),
        out_shape=(s, s, s, s),
    )(y, labels2d, feas_parts)


def kernel(x, edge_index, y_carry, z_carry, labels, H_step, params):
    p = params
    src = edge_index[0]
    dst = edge_index[1]
    zeros = jnp.zeros((RPT, H), jnp.float32)
    pad = EP - E
    pad_dst = N + (jnp.arange(pad, dtype=jnp.int32) % (NP - N))
    pad_src = jnp.arange(pad, dtype=jnp.int32) % N
    srcp = jnp.concatenate([src, pad_src])
    dstp = jnp.concatenate([dst, pad_dst])

    wp = p["latent_proj_w"]
    wpx, wpy, wpz = wp[:H], wp[H:H + 1], wp[H + 1:]
    bp = p["latent_proj_b"].reshape(1, H)
    lng, lnb = p["latent_norm_g"].reshape(1, H), p["latent_norm_b"].reshape(1, H)

    wo = p["output_proj_w"]
    woy, woz = wo[:1], wo[1:]
    bo = p["output_norm_b"]

    c0 = _embed(x, p["x_embed_w"], p["x_embed_b"].reshape(1, H),
                p["x_norm_g"].reshape(1, H), p["x_norm_b"].reshape(1, H), wpx)

    seg = _seg_sum_kernel()
    feask = _feas_kernel()

    gins = []
    for gp in p["gin"]:
        gins.append((
            gp["eps"].reshape(1, 1),
            gp["w1"], gp["b1"].reshape(1, 2 * H),
            gp["ln_g"].reshape(1, 2 * H), gp["ln_b"].reshape(1, 2 * H),
            gp["w2"], gp["b2"].reshape(1, H),
            gp["post_ln_g"].reshape(1, H), gp["post_ln_b"].reshape(1, H),
        ))

    y, z = y_carry, z_carry
    L_CYCLES, H_CYCLES = 6, 3
    for _ in range(H_CYCLES):
        for _ in range(L_CYCLES):
            h = _stepin(c0, y, z, wpy, wpz, bp, lng, lnb)
            for (eps, w1, b1, g1, bb1, w2, b2, pg, pb) in gins:
                parts = seg(srcp, dstp, h, zeros)
                h = _gin_post(h, parts[:N], parts[NP:NP + N], eps,
                              w1, b1, g1, bb1, w2, b2, pg, pb)
            z = h
        y = _outstep(y, z, woy, woz, p["output_proj_b"].reshape(1, H),
                     p["output_norm_g"].reshape(1, H),
                     p["output_norm_b"].reshape(1, H),
                     p["head_w1"], p["head_b1"].reshape(1, H),
                     p["head_w2"], p["head_b2"].reshape(1, 1))

    probs = _probs(y)
    feas_parts = feask(probs.reshape(-1), src, dst)
    a, b, pcnt, fsum = _loss(y, labels.reshape(N, 1), feas_parts.reshape(1, NW * 16))

    pos = jnp.clip(pcnt[0, 0], 1.0, None)
    neg = jnp.clip(float(N) - pos, 1.0, None)
    pw = neg / pos
    bce = (pw * a[0, 0] + b[0, 0]) / float(N)
    feas = fsum[0, 0] / float(E)
    return bce + 50.0 * feas


# speedup vs baseline: 3.2859x; 1.1584x over previous
import functools

import jax
import jax.numpy as jnp
from jax import lax
from jax.experimental import pallas as pl
from jax.experimental.pallas import tpu as pltpu
from jax.experimental.pallas import tpu_sc as plsc

N = 10000
E = 320000
H = 128
NC = 2
NS = 16
NW = NC * NS
C = 128
NCHUNK = E // C
BASE_CHUNKS = NCHUNK // NW
EXTRA = NCHUNK - BASE_CHUNKS * NW
NP = 10240
RPT = NP // NS

R = 1000
GRID = N // R


def _mesh():
    return plsc.VectorSubcoreMesh(
        core_axis_name="c", subcore_axis_name="s", num_cores=NC, num_subcores=NS
    )


EP = 327680
CPW = EP // C // NW
IB = 8
GROUPS = CPW // IB


@functools.lru_cache(maxsize=None)
def _seg_sum_kernel():
    @functools.partial(
        pl.kernel,
        out_type=jax.ShapeDtypeStruct((2 * NP, H), jnp.float32),
        mesh=_mesh(),
        scratch_types=[
            pltpu.VMEM((IB, C), jnp.int32),
            pltpu.VMEM((IB, C), jnp.int32),
            pltpu.VMEM((C, H), jnp.float32),
            pltpu.VMEM((C, H), jnp.float32),
            pltpu.VMEM_SHARED((NP, H), jnp.float32),
            pltpu.SemaphoreType.DMA,
            pltpu.SemaphoreType.DMA,
            pltpu.SemaphoreType.DMA,
            pltpu.SemaphoreType.DMA,
        ],
        name="sc_seg_sum",
    )
    def k(src_hbm, dst_hbm, h_hbm, zeros_hbm, out_hbm,
          srcb_v, dstb_v, rows0_v, rows1_v,
          part_s, sem_g0, sem_g1, sem_s0, sem_s1):
        cid = lax.axis_index("c")
        sid = lax.axis_index("s")
        w = sid * NC + cid
        chunk0 = w * CPW

        pltpu.sync_copy(zeros_hbm.at[pl.ds(sid * RPT, RPT)],
                        part_s.at[pl.ds(sid * RPT, RPT)])
        plsc.subcore_barrier()

        rows = (rows0_v, rows1_v)
        sem_g = (sem_g0, sem_g1)
        sem_s = (sem_s0, sem_s1)

        ngroups = (CPW // IB) + (w < 0).astype(jnp.int32)

        def body(t, carry):
            c0 = chunk0 + t * IB
            pltpu.sync_copy(src_hbm.at[pl.ds(c0, IB)], srcb_v)
            pltpu.sync_copy(dst_hbm.at[pl.ds(c0, IB)], dstb_v)
            g = [None] * IB
            sct = [None] * IB
            g[0] = pltpu.async_copy(h_hbm.at[srcb_v.at[0]], rows[0], sem_g[0])
            g[1] = pltpu.async_copy(h_hbm.at[srcb_v.at[1]], rows[1], sem_g[1])
            for j in range(IB):
                p = j % 2
                g[j].wait()
                sct[j] = pltpu.async_copy(rows[p], part_s.at[dstb_v.at[j]],
                                          sem_s[p], add=True)
                if j + 2 < IB:
                    sct[j].wait()
                    g[j + 2] = pltpu.async_copy(h_hbm.at[srcb_v.at[j + 2]],
                                                rows[p], sem_g[p])
            sct[IB - 2].wait()
            sct[IB - 1].wait()
            return carry

        lax.fori_loop(0, ngroups, body, 0)
        plsc.subcore_barrier()
        row0 = cid * NP + sid * RPT
        pltpu.sync_copy(part_s.at[pl.ds(sid * RPT, RPT)],
                        out_hbm.at[pl.ds(row0, RPT)])

    return k


@functools.lru_cache(maxsize=None)
def _feas_kernel():
    @functools.partial(
        pl.kernel,
        out_type=jax.ShapeDtypeStruct((NW * 16,), jnp.float32),
        mesh=_mesh(),
        scratch_types=[
            pltpu.VMEM((C,), jnp.int32),
            pltpu.VMEM((C,), jnp.int32),
            pltpu.VMEM((C,), jnp.float32),
            pltpu.VMEM((C,), jnp.float32),
            pltpu.VMEM((16,), jnp.float32),
            pltpu.SemaphoreType.DMA,
        ],
        name="sc_feas",
    )
    def k(probs_hbm, src_hbm, dst_hbm, out_hbm,
          si_v, di_v, sv_v, dv_v, acc_v, sem):
        cid = lax.axis_index("c")
        sid = lax.axis_index("s")
        w = sid * NC + cid
        nchunks = BASE_CHUNKS + (w < EXTRA).astype(jnp.int32)

        def body(i, acc):
            base = (w + i * NW) * C
            pltpu.sync_copy(src_hbm.at[pl.ds(base, C)], si_v)
            pltpu.sync_copy(dst_hbm.at[pl.ds(base, C)], di_v)
            pltpu.async_copy(probs_hbm.at[si_v], sv_v, sem).wait()
            pltpu.async_copy(probs_hbm.at[di_v], dv_v, sem).wait()
            for j in range(C // 16):
                acc = acc + sv_v[pl.ds(j * 16, 16)] * dv_v[pl.ds(j * 16, 16)]
            return acc

        acc = lax.fori_loop(0, nchunks, body, jnp.zeros((16,), jnp.float32))
        acc_v[...] = acc
        pltpu.sync_copy(acc_v, out_hbm.at[pl.ds(w * 16, 16)])

    return k


def _ln(t, g, b, eps=1e-5):
    m = jnp.mean(t, axis=-1, keepdims=True)
    tc = t - m
    v = jnp.mean(tc * tc, axis=-1, keepdims=True)
    return tc * lax.rsqrt(v + eps) * g + b


def _full(shape):
    return pl.BlockSpec(shape, lambda i: (0,) * len(shape))


def _rows(width):
    return pl.BlockSpec((R, width), lambda i: (i, 0))


def _dot(a, b):
    return jnp.dot(a, b, preferred_element_type=jnp.float32)


def _embed_body(x_ref, xw_ref, xb_ref, g_ref, b_ref, wpx_ref, c0_ref):
    xx = x_ref[...]
    e = xx[:, 0:1] * xw_ref[0:1, :] + xx[:, 1:2] * xw_ref[1:2, :] + xb_ref[...]
    e = _ln(e, g_ref[...], b_ref[...])
    c0_ref[...] = _dot(e, wpx_ref[...])


def _embed(x, xw, xb, g, b, wpx):
    return pl.pallas_call(
        _embed_body,
        grid=(GRID,),
        in_specs=[_rows(2), _full((2, H)), _full((1, H)), _full((1, H)),
                  _full((1, H)), _full((H, H))],
        out_specs=_rows(H),
        out_shape=jax.ShapeDtypeStruct((N, H), jnp.float32),
    )(x, xw, xb, g, b, wpx)


def _stepin_body(c0_ref, y_ref, z_ref, wpy_ref, wpz_ref, bp_ref, g_ref, b_ref,
                 h_ref):
    t = (c0_ref[...] + jax.nn.sigmoid(y_ref[...]) * wpy_ref[...]
         + _dot(z_ref[...], wpz_ref[...]) + bp_ref[...])
    h_ref[...] = _ln(t, g_ref[...], b_ref[...])


def _stepin(c0, y, z, wpy, wpz, bp, g, b):
    return pl.pallas_call(
        _stepin_body,
        grid=(GRID,),
        in_specs=[_rows(H), _rows(1), _rows(H), _full((1, H)), _full((H, H)),
                  _full((1, H)), _full((1, H)), _full((1, H))],
        out_specs=_rows(H),
        out_shape=jax.ShapeDtypeStruct((N, H), jnp.float32),
    )(c0, y, z, wpy, wpz, bp, g, b)


def _gin_body(h_ref, p0_ref, p1_ref, eps_ref, w1_ref, b1_ref, g1_ref, bb1_ref,
              w2_ref, b2_ref, pg_ref, pb_ref, out_ref):
    h = h_ref[...]
    u = (1.0 + eps_ref[0, 0]) * h + p0_ref[...] + p1_ref[...]
    t = _dot(u, w1_ref[...]) + b1_ref[...]
    t = jax.nn.gelu(_ln(t, g1_ref[...], bb1_ref[...]))
    v = _dot(t, w2_ref[...]) + b2_ref[...]
    out_ref[...] = _ln(h + jax.nn.gelu(v), pg_ref[...], pb_ref[...])


def _gin_post(h, p0, p1, eps, w1, b1, g1, bb1, w2, b2, pg, pb):
    return pl.pallas_call(
        _gin_body,
        grid=(GRID,),
        in_specs=[_rows(H), _rows(H), _rows(H),
                  pl.BlockSpec(memory_space=pltpu.SMEM),
                  _full((H, 2 * H)), _full((1, 2 * H)), _full((1, 2 * H)),
                  _full((1, 2 * H)), _full((2 * H, H)), _full((1, H)),
                  _full((1, H)), _full((1, H))],
        out_specs=_rows(H),
        out_shape=jax.ShapeDtypeStruct((N, H), jnp.float32),
    )(h, p0, p1, eps, w1, b1, g1, bb1, w2, b2, pg, pb)


def _outstep_body(y_ref, z_ref, woy_ref, woz_ref, bo_ref, og_ref, ob_ref,
                  w1_ref, b1_ref, w2_ref, b2_ref, yo_ref):
    t = (y_ref[...] * woy_ref[...] + _dot(z_ref[...], woz_ref[...])
         + bo_ref[...])
    t = _ln(t, og_ref[...], ob_ref[...])
    g = jax.nn.gelu(_dot(t, w1_ref[...]) + b1_ref[...])
    yo_ref[...] = _dot(g, w2_ref[...]) + b2_ref[0, 0]


def _outstep(y, z, woy, woz, bo, og, ob, w1, b1, w2, b2):
    return pl.pallas_call(
        _outstep_body,
        grid=(GRID,),
        in_specs=[_rows(1), _rows(H), _full((1, H)), _full((H, H)),
                  _full((1, H)), _full((1, H)), _full((1, H)),
                  _full((H, H)), _full((1, H)), _full((H, 1)),
                  pl.BlockSpec(memory_space=pltpu.SMEM)],
        out_specs=_rows(1),
        out_shape=jax.ShapeDtypeStruct((N, 1), jnp.float32),
    )(y, z, woy, woz, bo, og, ob, w1, b1, w2, b2)


def _probs_body(y_ref, p_ref):
    p_ref[...] = jax.nn.sigmoid(jnp.clip(y_ref[...], -10.0, 10.0))


def _probs(y):
    return pl.pallas_call(
        _probs_body,
        grid=(GRID,),
        in_specs=[_rows(1)],
        out_specs=_rows(1),
        out_shape=jax.ShapeDtypeStruct((N, 1), jnp.float32),
    )(y)


def _loss_body(y_ref, lab_ref, fp_ref, a_ref, b_ref, p_ref, f_ref):
    i = pl.program_id(0)
    l = jnp.clip(y_ref[...], -10.0, 10.0)
    lab = lab_ref[...].astype(jnp.float32)
    a = jnp.sum(lab * jax.nn.softplus(-l))
    b = jnp.sum((1.0 - lab) * jax.nn.softplus(l))
    p = jnp.sum(lab)

    @pl.when(i == 0)
    def _():
        zz = jnp.zeros((1, 1), jnp.float32)
        a_ref[...] = zz
        b_ref[...] = zz
        p_ref[...] = zz
        f_ref[...] = jnp.sum(fp_ref[...]).reshape(1, 1)

    a_ref[...] = a_ref[...] + a
    b_ref[...] = b_ref[...] + b
    p_ref[...] = p_ref[...] + p


def _loss(y, labels2d, feas_parts):
    s = jax.ShapeDtypeStruct((1, 1), jnp.float32)
    one = pl.BlockSpec((1, 1), lambda i: (0, 0))
    return pl.pallas_call(
        _loss_body,
        grid=(GRID,),
        in_specs=[_rows(1), _rows(1), pl.BlockSpec((1, NW * 16), lambda i: (0, 0))],
        out_specs=(one, one, one, one),
        out_shape=(s, s, s, s),
    )(y, labels2d, feas_parts)


def kernel(x, edge_index, y_carry, z_carry, labels, H_step, params):
    p = params
    src = edge_index[0]
    dst = edge_index[1]
    zeros = jnp.zeros((NP, H), jnp.float32)
    pad = EP - E
    pad_dst = N + (jnp.arange(pad, dtype=jnp.int32) % (NP - N))
    pad_src = jnp.arange(pad, dtype=jnp.int32) % N
    srcp = jnp.concatenate([src, pad_src]).reshape(-1, C)
    dstp = jnp.concatenate([dst, pad_dst]).reshape(-1, C)

    wp = p["latent_proj_w"]
    wpx, wpy, wpz = wp[:H], wp[H:H + 1], wp[H + 1:]
    bp = p["latent_proj_b"].reshape(1, H)
    lng, lnb = p["latent_norm_g"].reshape(1, H), p["latent_norm_b"].reshape(1, H)

    wo = p["output_proj_w"]
    woy, woz = wo[:1], wo[1:]
    bo = p["output_norm_b"]

    c0 = _embed(x, p["x_embed_w"], p["x_embed_b"].reshape(1, H),
                p["x_norm_g"].reshape(1, H), p["x_norm_b"].reshape(1, H), wpx)

    seg = _seg_sum_kernel()
    feask = _feas_kernel()

    gins = []
    for gp in p["gin"]:
        gins.append((
            gp["eps"].reshape(1, 1),
            gp["w1"], gp["b1"].reshape(1, 2 * H),
            gp["ln_g"].reshape(1, 2 * H), gp["ln_b"].reshape(1, 2 * H),
            gp["w2"], gp["b2"].reshape(1, H),
            gp["post_ln_g"].reshape(1, H), gp["post_ln_b"].reshape(1, H),
        ))

    y, z = y_carry, z_carry
    L_CYCLES, H_CYCLES = 6, 3
    for _ in range(H_CYCLES):
        for _ in range(L_CYCLES):
            h = _stepin(c0, y, z, wpy, wpz, bp, lng, lnb)
            for (eps, w1, b1, g1, bb1, w2, b2, pg, pb) in gins:
                parts = seg(srcp, dstp, h, zeros)
                h = _gin_post(h, parts[:N], parts[NP:NP + N], eps,
                              w1, b1, g1, bb1, w2, b2, pg, pb)
            z = h
        y = _outstep(y, z, woy, woz, p["output_proj_b"].reshape(1, H),
                     p["output_norm_g"].reshape(1, H),
                     p["output_norm_b"].reshape(1, H),
                     p["head_w1"], p["head_b1"].reshape(1, H),
                     p["head_w2"], p["head_b2"].reshape(1, 1))

    probs = _probs(y)
    feas_parts = feask(probs.reshape(-1), src, dst)
    a, b, pcnt, fsum = _loss(y, labels.reshape(N, 1), feas_parts.reshape(1, NW * 16))

    pos = jnp.clip(pcnt[0, 0], 1.0, None)
    neg = jnp.clip(float(N) - pos, 1.0, None)
    pw = neg / pos
    bce = (pw * a[0, 0] + b[0, 0]) / float(N)
    feas = fsum[0, 0] / float(E)
    return bce + 50.0 * feas


# speedup vs baseline: 3.5315x; 1.0747x over previous
import functools

import jax
import jax.numpy as jnp
from jax import lax
from jax.experimental import pallas as pl
from jax.experimental.pallas import tpu as pltpu
from jax.experimental.pallas import tpu_sc as plsc

N = 10000
E = 320000
H = 128
NC = 2
NS = 16
NW = NC * NS
C = 128
NCHUNK = E // C
BASE_CHUNKS = NCHUNK // NW
EXTRA = NCHUNK - BASE_CHUNKS * NW
NP = 10240
RPT = NP // NS

R = 1000
GRID = N // R


def _mesh():
    return plsc.VectorSubcoreMesh(
        core_axis_name="c", subcore_axis_name="s", num_cores=NC, num_subcores=NS
    )


EP = 327680
CPW = EP // C // NW
IB = 16
GROUPS = CPW // IB


@functools.lru_cache(maxsize=None)
def _seg_sum_kernel():
    @functools.partial(
        pl.kernel,
        out_type=jax.ShapeDtypeStruct((2 * NP, H), jnp.float32),
        mesh=_mesh(),
        scratch_types=[
            pltpu.VMEM((IB, C), jnp.int32),
            pltpu.VMEM((IB, C), jnp.int32),
            pltpu.VMEM((C, H), jnp.float32),
            pltpu.VMEM((C, H), jnp.float32),
            pltpu.VMEM_SHARED((NP, H), jnp.float32),
            pltpu.SemaphoreType.DMA,
            pltpu.SemaphoreType.DMA,
            pltpu.SemaphoreType.DMA,
            pltpu.SemaphoreType.DMA,
        ],
        name="sc_seg_sum",
    )
    def k(src_hbm, dst_hbm, h_hbm, zeros_hbm, out_hbm,
          srcb_v, dstb_v, rows0_v, rows1_v,
          part_s, sem_g0, sem_g1, sem_s0, sem_s1):
        cid = lax.axis_index("c")
        sid = lax.axis_index("s")
        w = sid * NC + cid
        chunk0 = w * CPW

        pltpu.sync_copy(zeros_hbm.at[pl.ds(sid * RPT, RPT)],
                        part_s.at[pl.ds(sid * RPT, RPT)])
        plsc.subcore_barrier()

        rows = (rows0_v, rows1_v)
        sem_g = (sem_g0, sem_g1)
        sem_s = (sem_s0, sem_s1)

        ngroups = (CPW // IB) + (w < 0).astype(jnp.int32)

        def body(t, carry):
            c0 = chunk0 + t * IB
            pltpu.sync_copy(src_hbm.at[pl.ds(c0, IB)], srcb_v)
            pltpu.sync_copy(dst_hbm.at[pl.ds(c0, IB)], dstb_v)
            g = [None] * IB
            sct = [None] * IB
            g[0] = pltpu.async_copy(h_hbm.at[srcb_v.at[0]], rows[0], sem_g[0])
            g[1] = pltpu.async_copy(h_hbm.at[srcb_v.at[1]], rows[1], sem_g[1])
            for j in range(IB):
                p = j % 2
                g[j].wait()
                sct[j] = pltpu.async_copy(rows[p], part_s.at[dstb_v.at[j]],
                                          sem_s[p], add=True)
                if j + 2 < IB:
                    sct[j].wait()
                    g[j + 2] = pltpu.async_copy(h_hbm.at[srcb_v.at[j + 2]],
                                                rows[p], sem_g[p])
            sct[IB - 2].wait()
            sct[IB - 1].wait()
            return carry

        lax.fori_loop(0, ngroups, body, 0)
        plsc.subcore_barrier()
        row0 = cid * NP + sid * RPT
        pltpu.sync_copy(part_s.at[pl.ds(sid * RPT, RPT)],
                        out_hbm.at[pl.ds(row0, RPT)])

    return k


@functools.lru_cache(maxsize=None)
def _feas_kernel():
    @functools.partial(
        pl.kernel,
        out_type=jax.ShapeDtypeStruct((NW * 16,), jnp.float32),
        mesh=_mesh(),
        scratch_types=[
            pltpu.VMEM((C,), jnp.int32),
            pltpu.VMEM((C,), jnp.int32),
            pltpu.VMEM((C,), jnp.float32),
            pltpu.VMEM((C,), jnp.float32),
            pltpu.VMEM((16,), jnp.float32),
            pltpu.SemaphoreType.DMA,
        ],
        name="sc_feas",
    )
    def k(probs_hbm, src_hbm, dst_hbm, out_hbm,
          si_v, di_v, sv_v, dv_v, acc_v, sem):
        cid = lax.axis_index("c")
        sid = lax.axis_index("s")
        w = sid * NC + cid
        nchunks = BASE_CHUNKS + (w < EXTRA).astype(jnp.int32)

        def body(i, acc):
            base = (w + i * NW) * C
            pltpu.sync_copy(src_hbm.at[pl.ds(base, C)], si_v)
            pltpu.sync_copy(dst_hbm.at[pl.ds(base, C)], di_v)
            pltpu.async_copy(probs_hbm.at[si_v], sv_v, sem).wait()
            pltpu.async_copy(probs_hbm.at[di_v], dv_v, sem).wait()
            for j in range(C // 16):
                acc = acc + sv_v[pl.ds(j * 16, 16)] * dv_v[pl.ds(j * 16, 16)]
            return acc

        acc = lax.fori_loop(0, nchunks, body, jnp.zeros((16,), jnp.float32))
        acc_v[...] = acc
        pltpu.sync_copy(acc_v, out_hbm.at[pl.ds(w * 16, 16)])

    return k


def _ln(t, g, b, eps=1e-5):
    m = jnp.mean(t, axis=-1, keepdims=True)
    tc = t - m
    v = jnp.mean(tc * tc, axis=-1, keepdims=True)
    return tc * lax.rsqrt(v + eps) * g + b


def _full(shape):
    return pl.BlockSpec(shape, lambda i: (0,) * len(shape))


def _rows(width):
    return pl.BlockSpec((R, width), lambda i: (i, 0))


def _dot(a, b):
    return jnp.dot(a, b, preferred_element_type=jnp.float32)


def _embed_body(x_ref, xw_ref, xb_ref, g_ref, b_ref, wpx_ref, c0_ref):
    xx = x_ref[...]
    e = xx[:, 0:1] * xw_ref[0:1, :] + xx[:, 1:2] * xw_ref[1:2, :] + xb_ref[...]
    e = _ln(e, g_ref[...], b_ref[...])
    c0_ref[...] = _dot(e, wpx_ref[...])


def _embed(x, xw, xb, g, b, wpx):
    return pl.pallas_call(
        _embed_body,
        grid=(GRID,),
        in_specs=[_rows(2), _full((2, H)), _full((1, H)), _full((1, H)),
                  _full((1, H)), _full((H, H))],
        out_specs=_rows(H),
        out_shape=jax.ShapeDtypeStruct((N, H), jnp.float32),
    )(x, xw, xb, g, b, wpx)


def _stepin_body(c0_ref, y_ref, z_ref, wpy_ref, wpz_ref, bp_ref, g_ref, b_ref,
                 h_ref):
    t = (c0_ref[...] + jax.nn.sigmoid(y_ref[...]) * wpy_ref[...]
         + _dot(z_ref[...], wpz_ref[...]) + bp_ref[...])
    h_ref[...] = _ln(t, g_ref[...], b_ref[...])


def _stepin(c0, y, z, wpy, wpz, bp, g, b):
    return pl.pallas_call(
        _stepin_body,
        grid=(GRID,),
        in_specs=[_rows(H), _rows(1), _rows(H), _full((1, H)), _full((H, H)),
                  _full((1, H)), _full((1, H)), _full((1, H))],
        out_specs=_rows(H),
        out_shape=jax.ShapeDtypeStruct((N, H), jnp.float32),
    )(c0, y, z, wpy, wpz, bp, g, b)


def _gin_body(h_ref, p0_ref, p1_ref, eps_ref, w1_ref, b1_ref, g1_ref, bb1_ref,
              w2_ref, b2_ref, pg_ref, pb_ref, out_ref):
    h = h_ref[...]
    u = (1.0 + eps_ref[0, 0]) * h + p0_ref[...] + p1_ref[...]
    t = _dot(u, w1_ref[...]) + b1_ref[...]
    t = jax.nn.gelu(_ln(t, g1_ref[...], bb1_ref[...]))
    v = _dot(t, w2_ref[...]) + b2_ref[...]
    out_ref[...] = _ln(h + jax.nn.gelu(v), pg_ref[...], pb_ref[...])


def _gin_post(h, p0, p1, eps, w1, b1, g1, bb1, w2, b2, pg, pb):
    return pl.pallas_call(
        _gin_body,
        grid=(GRID,),
        in_specs=[_rows(H), _rows(H), _rows(H),
                  pl.BlockSpec(memory_space=pltpu.SMEM),
                  _full((H, 2 * H)), _full((1, 2 * H)), _full((1, 2 * H)),
                  _full((1, 2 * H)), _full((2 * H, H)), _full((1, H)),
                  _full((1, H)), _full((1, H))],
        out_specs=_rows(H),
        out_shape=jax.ShapeDtypeStruct((N, H), jnp.float32),
    )(h, p0, p1, eps, w1, b1, g1, bb1, w2, b2, pg, pb)


def _outstep_body(y_ref, z_ref, woy_ref, woz_ref, bo_ref, og_ref, ob_ref,
                  w1_ref, b1_ref, w2_ref, b2_ref, yo_ref):
    t = (y_ref[...] * woy_ref[...] + _dot(z_ref[...], woz_ref[...])
         + bo_ref[...])
    t = _ln(t, og_ref[...], ob_ref[...])
    g = jax.nn.gelu(_dot(t, w1_ref[...]) + b1_ref[...])
    yo_ref[...] = _dot(g, w2_ref[...]) + b2_ref[0, 0]


def _outstep(y, z, woy, woz, bo, og, ob, w1, b1, w2, b2):
    return pl.pallas_call(
        _outstep_body,
        grid=(GRID,),
        in_specs=[_rows(1), _rows(H), _full((1, H)), _full((H, H)),
                  _full((1, H)), _full((1, H)), _full((1, H)),
                  _full((H, H)), _full((1, H)), _full((H, 1)),
                  pl.BlockSpec(memory_space=pltpu.SMEM)],
        out_specs=_rows(1),
        out_shape=jax.ShapeDtypeStruct((N, 1), jnp.float32),
    )(y, z, woy, woz, bo, og, ob, w1, b1, w2, b2)


def _probs_body(y_ref, p_ref):
    p_ref[...] = jax.nn.sigmoid(jnp.clip(y_ref[...], -10.0, 10.0))


def _probs(y):
    return pl.pallas_call(
        _probs_body,
        grid=(GRID,),
        in_specs=[_rows(1)],
        out_specs=_rows(1),
        out_shape=jax.ShapeDtypeStruct((N, 1), jnp.float32),
    )(y)


def _loss_body(y_ref, lab_ref, fp_ref, a_ref, b_ref, p_ref, f_ref):
    i = pl.program_id(0)
    l = jnp.clip(y_ref[...], -10.0, 10.0)
    lab = lab_ref[...].astype(jnp.float32)
    a = jnp.sum(lab * jax.nn.softplus(-l))
    b = jnp.sum((1.0 - lab) * jax.nn.softplus(l))
    p = jnp.sum(lab)

    @pl.when(i == 0)
    def _():
        zz = jnp.zeros((1, 1), jnp.float32)
        a_ref[...] = zz
        b_ref[...] = zz
        p_ref[...] = zz
        f_ref[...] = jnp.sum(fp_ref[...]).reshape(1, 1)

    a_ref[...] = a_ref[...] + a
    b_ref[...] = b_ref[...] + b
    p_ref[...] = p_ref[...] + p


def _loss(y, labels2d, feas_parts):
    s = jax.ShapeDtypeStruct((1, 1), jnp.float32)
    one = pl.BlockSpec((1, 1), lambda i: (0, 0))
    return pl.pallas_call(
        _loss_body,
        grid=(GRID,),
        in_specs=[_rows(1), _rows(1), pl.BlockSpec((1, NW * 16), lambda i: (0, 0))],
        out_specs=(one, one, one, one),
        out_shape=(s, s, s, s),
    )(y, labels2d, feas_parts)


def kernel(x, edge_index, y_carry, z_carry, labels, H_step, params):
    p = params
    src = edge_index[0]
    dst = edge_index[1]
    zeros = jnp.zeros((NP, H), jnp.float32)
    pad = EP - E
    pad_dst = N + (jnp.arange(pad, dtype=jnp.int32) % (NP - N))
    pad_src = jnp.arange(pad, dtype=jnp.int32) % N
    srcp = jnp.concatenate([src, pad_src]).reshape(-1, C)
    dstp = jnp.concatenate([dst, pad_dst]).reshape(-1, C)

    wp = p["latent_proj_w"]
    wpx, wpy, wpz = wp[:H], wp[H:H + 1], wp[H + 1:]
    bp = p["latent_proj_b"].reshape(1, H)
    lng, lnb = p["latent_norm_g"].reshape(1, H), p["latent_norm_b"].reshape(1, H)

    wo = p["output_proj_w"]
    woy, woz = wo[:1], wo[1:]
    bo = p["output_norm_b"]

    c0 = _embed(x, p["x_embed_w"], p["x_embed_b"].reshape(1, H),
                p["x_norm_g"].reshape(1, H), p["x_norm_b"].reshape(1, H), wpx)

    seg = _seg_sum_kernel()
    feask = _feas_kernel()

    gins = []
    for gp in p["gin"]:
        gins.append((
            gp["eps"].reshape(1, 1),
            gp["w1"], gp["b1"].reshape(1, 2 * H),
            gp["ln_g"].reshape(1, 2 * H), gp["ln_b"].reshape(1, 2 * H),
            gp["w2"], gp["b2"].reshape(1, H),
            gp["post_ln_g"].reshape(1, H), gp["post_ln_b"].reshape(1, H),
        ))

    y, z = y_carry, z_carry
    L_CYCLES, H_CYCLES = 6, 3
    for _ in range(H_CYCLES):
        for _ in range(L_CYCLES):
            h = _stepin(c0, y, z, wpy, wpz, bp, lng, lnb)
            for (eps, w1, b1, g1, bb1, w2, b2, pg, pb) in gins:
                parts = seg(srcp, dstp, h, zeros)
                h = _gin_post(h, parts[:N], parts[NP:NP + N], eps,
                              w1, b1, g1, bb1, w2, b2, pg, pb)
            z = h
        y = _outstep(y, z, woy, woz, p["output_proj_b"].reshape(1, H),
                     p["output_norm_g"].reshape(1, H),
                     p["output_norm_b"].reshape(1, H),
                     p["head_w1"], p["head_b1"].reshape(1, H),
                     p["head_w2"], p["head_b2"].reshape(1, 1))

    probs = _probs(y)
    feas_parts = feask(probs.reshape(-1), src, dst)
    a, b, pcnt, fsum = _loss(y, labels.reshape(N, 1), feas_parts.reshape(1, NW * 16))

    pos = jnp.clip(pcnt[0, 0], 1.0, None)
    neg = jnp.clip(float(N) - pos, 1.0, None)
    pw = neg / pos
    bce = (pw * a[0, 0] + b[0, 0]) / float(N)
    feas = fsum[0, 0] / float(E)
    return bce + 50.0 * feas


# speedup vs baseline: 3.7067x; 1.0496x over previous
import functools

import jax
import jax.numpy as jnp
from jax import lax
from jax.experimental import pallas as pl
from jax.experimental.pallas import tpu as pltpu
from jax.experimental.pallas import tpu_sc as plsc

N = 10000
E = 320000
H = 128
NC = 2
NS = 16
NW = NC * NS
C = 128
NCHUNK = E // C
BASE_CHUNKS = NCHUNK // NW
EXTRA = NCHUNK - BASE_CHUNKS * NW
NP = 10240
RPT = NP // NS

R = 1000
GRID = N // R


def _mesh():
    return plsc.VectorSubcoreMesh(
        core_axis_name="c", subcore_axis_name="s", num_cores=NC, num_subcores=NS
    )


EP = 327680
CPW = EP // C // NW
IB = 40
GROUPS = CPW // IB


@functools.lru_cache(maxsize=None)
def _seg_sum_kernel():
    @functools.partial(
        pl.kernel,
        out_type=jax.ShapeDtypeStruct((2 * NP, H), jnp.float32),
        mesh=_mesh(),
        scratch_types=[
            pltpu.VMEM((IB, C), jnp.int32),
            pltpu.VMEM((IB, C), jnp.int32),
            pltpu.VMEM((C, H), jnp.float32),
            pltpu.VMEM((C, H), jnp.float32),
            pltpu.VMEM_SHARED((NP, H), jnp.float32),
            pltpu.SemaphoreType.DMA,
            pltpu.SemaphoreType.DMA,
            pltpu.SemaphoreType.DMA,
            pltpu.SemaphoreType.DMA,
        ],
        name="sc_seg_sum",
    )
    def k(src_hbm, dst_hbm, h_hbm, zeros_hbm, out_hbm,
          srcb_v, dstb_v, rows0_v, rows1_v,
          part_s, sem_g0, sem_g1, sem_s0, sem_s1):
        cid = lax.axis_index("c")
        sid = lax.axis_index("s")
        w = sid * NC + cid
        chunk0 = w * CPW

        pltpu.sync_copy(zeros_hbm.at[pl.ds(sid * RPT, RPT)],
                        part_s.at[pl.ds(sid * RPT, RPT)])
        plsc.subcore_barrier()

        rows = (rows0_v, rows1_v)
        sem_g = (sem_g0, sem_g1)
        sem_s = (sem_s0, sem_s1)

        ngroups = (CPW // IB) + (w < 0).astype(jnp.int32)

        def body(t, carry):
            c0 = chunk0 + t * IB
            pltpu.sync_copy(src_hbm.at[pl.ds(c0, IB)], srcb_v)
            pltpu.sync_copy(dst_hbm.at[pl.ds(c0, IB)], dstb_v)
            g = [None] * IB
            sct = [None] * IB
            g[0] = pltpu.async_copy(h_hbm.at[srcb_v.at[0]], rows[0], sem_g[0])
            g[1] = pltpu.async_copy(h_hbm.at[srcb_v.at[1]], rows[1], sem_g[1])
            for j in range(IB):
                p = j % 2
                g[j].wait()
                sct[j] = pltpu.async_copy(rows[p], part_s.at[dstb_v.at[j]],
                                          sem_s[p], add=True)
                if j + 2 < IB:
                    sct[j].wait()
                    g[j + 2] = pltpu.async_copy(h_hbm.at[srcb_v.at[j + 2]],
                                                rows[p], sem_g[p])
            sct[IB - 2].wait()
            sct[IB - 1].wait()
            return carry

        lax.fori_loop(0, ngroups, body, 0)
        plsc.subcore_barrier()
        row0 = cid * NP + sid * RPT
        pltpu.sync_copy(part_s.at[pl.ds(sid * RPT, RPT)],
                        out_hbm.at[pl.ds(row0, RPT)])

    return k


@functools.lru_cache(maxsize=None)
def _feas_kernel():
    @functools.partial(
        pl.kernel,
        out_type=jax.ShapeDtypeStruct((NW * 16,), jnp.float32),
        mesh=_mesh(),
        scratch_types=[
            pltpu.VMEM((C,), jnp.int32),
            pltpu.VMEM((C,), jnp.int32),
            pltpu.VMEM((C,), jnp.float32),
            pltpu.VMEM((C,), jnp.float32),
            pltpu.VMEM((16,), jnp.float32),
            pltpu.SemaphoreType.DMA,
        ],
        name="sc_feas",
    )
    def k(probs_hbm, src_hbm, dst_hbm, out_hbm,
          si_v, di_v, sv_v, dv_v, acc_v, sem):
        cid = lax.axis_index("c")
        sid = lax.axis_index("s")
        w = sid * NC + cid
        nchunks = BASE_CHUNKS + (w < EXTRA).astype(jnp.int32)

        def body(i, acc):
            base = (w + i * NW) * C
            pltpu.sync_copy(src_hbm.at[pl.ds(base, C)], si_v)
            pltpu.sync_copy(dst_hbm.at[pl.ds(base, C)], di_v)
            pltpu.async_copy(probs_hbm.at[si_v], sv_v, sem).wait()
            pltpu.async_copy(probs_hbm.at[di_v], dv_v, sem).wait()
            for j in range(C // 16):
                acc = acc + sv_v[pl.ds(j * 16, 16)] * dv_v[pl.ds(j * 16, 16)]
            return acc

        acc = lax.fori_loop(0, nchunks, body, jnp.zeros((16,), jnp.float32))
        acc_v[...] = acc
        pltpu.sync_copy(acc_v, out_hbm.at[pl.ds(w * 16, 16)])

    return k


def _ln(t, g, b, eps=1e-5):
    m = jnp.mean(t, axis=-1, keepdims=True)
    tc = t - m
    v = jnp.mean(tc * tc, axis=-1, keepdims=True)
    return tc * lax.rsqrt(v + eps) * g + b


def _full(shape):
    return pl.BlockSpec(shape, lambda i: (0,) * len(shape))


def _rows(width):
    return pl.BlockSpec((R, width), lambda i: (i, 0))


def _dot(a, b):
    return jnp.dot(a, b, preferred_element_type=jnp.float32)


def _embed_body(x_ref, xw_ref, xb_ref, g_ref, b_ref, wpx_ref, c0_ref):
    xx = x_ref[...]
    e = xx[:, 0:1] * xw_ref[0:1, :] + xx[:, 1:2] * xw_ref[1:2, :] + xb_ref[...]
    e = _ln(e, g_ref[...], b_ref[...])
    c0_ref[...] = _dot(e, wpx_ref[...])


def _embed(x, xw, xb, g, b, wpx):
    return pl.pallas_call(
        _embed_body,
        grid=(GRID,),
        in_specs=[_rows(2), _full((2, H)), _full((1, H)), _full((1, H)),
                  _full((1, H)), _full((H, H))],
        out_specs=_rows(H),
        out_shape=jax.ShapeDtypeStruct((N, H), jnp.float32),
    )(x, xw, xb, g, b, wpx)


def _stepin_body(c0_ref, y_ref, z_ref, wpy_ref, wpz_ref, bp_ref, g_ref, b_ref,
                 h_ref):
    t = (c0_ref[...] + jax.nn.sigmoid(y_ref[...]) * wpy_ref[...]
         + _dot(z_ref[...], wpz_ref[...]) + bp_ref[...])
    h_ref[...] = _ln(t, g_ref[...], b_ref[...])


def _stepin(c0, y, z, wpy, wpz, bp, g, b):
    return pl.pallas_call(
        _stepin_body,
        grid=(GRID,),
        in_specs=[_rows(H), _rows(1), _rows(H), _full((1, H)), _full((H, H)),
                  _full((1, H)), _full((1, H)), _full((1, H))],
        out_specs=_rows(H),
        out_shape=jax.ShapeDtypeStruct((N, H), jnp.float32),
    )(c0, y, z, wpy, wpz, bp, g, b)


def _gin_body(h_ref, p0_ref, p1_ref, eps_ref, w1_ref, b1_ref, g1_ref, bb1_ref,
              w2_ref, b2_ref, pg_ref, pb_ref, out_ref):
    h = h_ref[...]
    u = (1.0 + eps_ref[0, 0]) * h + p0_ref[...] + p1_ref[...]
    t = _dot(u, w1_ref[...]) + b1_ref[...]
    t = jax.nn.gelu(_ln(t, g1_ref[...], bb1_ref[...]))
    v = _dot(t, w2_ref[...]) + b2_ref[...]
    out_ref[...] = _ln(h + jax.nn.gelu(v), pg_ref[...], pb_ref[...])


def _gin_post(h, p0, p1, eps, w1, b1, g1, bb1, w2, b2, pg, pb):
    return pl.pallas_call(
        _gin_body,
        grid=(GRID,),
        in_specs=[_rows(H), _rows(H), _rows(H),
                  pl.BlockSpec(memory_space=pltpu.SMEM),
                  _full((H, 2 * H)), _full((1, 2 * H)), _full((1, 2 * H)),
                  _full((1, 2 * H)), _full((2 * H, H)), _full((1, H)),
                  _full((1, H)), _full((1, H))],
        out_specs=_rows(H),
        out_shape=jax.ShapeDtypeStruct((N, H), jnp.float32),
    )(h, p0, p1, eps, w1, b1, g1, bb1, w2, b2, pg, pb)


def _outstep_body(y_ref, z_ref, woy_ref, woz_ref, bo_ref, og_ref, ob_ref,
                  w1_ref, b1_ref, w2_ref, b2_ref, yo_ref):
    t = (y_ref[...] * woy_ref[...] + _dot(z_ref[...], woz_ref[...])
         + bo_ref[...])
    t = _ln(t, og_ref[...], ob_ref[...])
    g = jax.nn.gelu(_dot(t, w1_ref[...]) + b1_ref[...])
    yo_ref[...] = _dot(g, w2_ref[...]) + b2_ref[0, 0]


def _outstep(y, z, woy, woz, bo, og, ob, w1, b1, w2, b2):
    return pl.pallas_call(
        _outstep_body,
        grid=(GRID,),
        in_specs=[_rows(1), _rows(H), _full((1, H)), _full((H, H)),
                  _full((1, H)), _full((1, H)), _full((1, H)),
                  _full((H, H)), _full((1, H)), _full((H, 1)),
                  pl.BlockSpec(memory_space=pltpu.SMEM)],
        out_specs=_rows(1),
        out_shape=jax.ShapeDtypeStruct((N, 1), jnp.float32),
    )(y, z, woy, woz, bo, og, ob, w1, b1, w2, b2)


def _probs_body(y_ref, p_ref):
    p_ref[...] = jax.nn.sigmoid(jnp.clip(y_ref[...], -10.0, 10.0))


def _probs(y):
    return pl.pallas_call(
        _probs_body,
        grid=(GRID,),
        in_specs=[_rows(1)],
        out_specs=_rows(1),
        out_shape=jax.ShapeDtypeStruct((N, 1), jnp.float32),
    )(y)


def _loss_body(y_ref, lab_ref, fp_ref, a_ref, b_ref, p_ref, f_ref):
    i = pl.program_id(0)
    l = jnp.clip(y_ref[...], -10.0, 10.0)
    lab = lab_ref[...].astype(jnp.float32)
    a = jnp.sum(lab * jax.nn.softplus(-l))
    b = jnp.sum((1.0 - lab) * jax.nn.softplus(l))
    p = jnp.sum(lab)

    @pl.when(i == 0)
    def _():
        zz = jnp.zeros((1, 1), jnp.float32)
        a_ref[...] = zz
        b_ref[...] = zz
        p_ref[...] = zz
        f_ref[...] = jnp.sum(fp_ref[...]).reshape(1, 1)

    a_ref[...] = a_ref[...] + a
    b_ref[...] = b_ref[...] + b
    p_ref[...] = p_ref[...] + p


def _loss(y, labels2d, feas_parts):
    s = jax.ShapeDtypeStruct((1, 1), jnp.float32)
    one = pl.BlockSpec((1, 1), lambda i: (0, 0))
    return pl.pallas_call(
        _loss_body,
        grid=(GRID,),
        in_specs=[_rows(1), _rows(1), pl.BlockSpec((1, NW * 16), lambda i: (0, 0))],
        out_specs=(one, one, one, one),
        out_shape=(s, s, s, s),
    )(y, labels2d, feas_parts)


def kernel(x, edge_index, y_carry, z_carry, labels, H_step, params):
    p = params
    src = edge_index[0]
    dst = edge_index[1]
    zeros = jnp.zeros((NP, H), jnp.float32)
    pad = EP - E
    pad_dst = N + (jnp.arange(pad, dtype=jnp.int32) % (NP - N))
    pad_src = jnp.arange(pad, dtype=jnp.int32) % N
    srcp = jnp.concatenate([src, pad_src]).reshape(-1, C)
    dstp = jnp.concatenate([dst, pad_dst]).reshape(-1, C)

    wp = p["latent_proj_w"]
    wpx, wpy, wpz = wp[:H], wp[H:H + 1], wp[H + 1:]
    bp = p["latent_proj_b"].reshape(1, H)
    lng, lnb = p["latent_norm_g"].reshape(1, H), p["latent_norm_b"].reshape(1, H)

    wo = p["output_proj_w"]
    woy, woz = wo[:1], wo[1:]
    bo = p["output_norm_b"]

    c0 = _embed(x, p["x_embed_w"], p["x_embed_b"].reshape(1, H),
                p["x_norm_g"].reshape(1, H), p["x_norm_b"].reshape(1, H), wpx)

    seg = _seg_sum_kernel()
    feask = _feas_kernel()

    gins = []
    for gp in p["gin"]:
        gins.append((
            gp["eps"].reshape(1, 1),
            gp["w1"], gp["b1"].reshape(1, 2 * H),
            gp["ln_g"].reshape(1, 2 * H), gp["ln_b"].reshape(1, 2 * H),
            gp["w2"], gp["b2"].reshape(1, H),
            gp["post_ln_g"].reshape(1, H), gp["post_ln_b"].reshape(1, H),
        ))

    y, z = y_carry, z_carry
    L_CYCLES, H_CYCLES = 6, 3
    for _ in range(H_CYCLES):
        for _ in range(L_CYCLES):
            h = _stepin(c0, y, z, wpy, wpz, bp, lng, lnb)
            for (eps, w1, b1, g1, bb1, w2, b2, pg, pb) in gins:
                parts = seg(srcp, dstp, h, zeros)
                h = _gin_post(h, parts[:N], parts[NP:NP + N], eps,
                              w1, b1, g1, bb1, w2, b2, pg, pb)
            z = h
        y = _outstep(y, z, woy, woz, p["output_proj_b"].reshape(1, H),
                     p["output_norm_g"].reshape(1, H),
                     p["output_norm_b"].reshape(1, H),
                     p["head_w1"], p["head_b1"].reshape(1, H),
                     p["head_w2"], p["head_b2"].reshape(1, 1))

    probs = _probs(y)
    feas_parts = feask(probs.reshape(-1), src, dst)
    a, b, pcnt, fsum = _loss(y, labels.reshape(N, 1), feas_parts.reshape(1, NW * 16))

    pos = jnp.clip(pcnt[0, 0], 1.0, None)
    neg = jnp.clip(float(N) - pos, 1.0, None)
    pw = neg / pos
    bce = (pw * a[0, 0] + b[0, 0]) / float(N)
    feas = fsum[0, 0] / float(E)
    return bce + 50.0 * feas


# speedup vs baseline: 3.8093x; 1.0277x over previous
import functools

import jax
import jax.numpy as jnp
from jax import lax
from jax.experimental import pallas as pl
from jax.experimental.pallas import tpu as pltpu
from jax.experimental.pallas import tpu_sc as plsc

N = 10000
E = 320000
H = 128
NC = 2
NS = 16
NW = NC * NS
C = 128
NCHUNK = E // C
BASE_CHUNKS = NCHUNK // NW
EXTRA = NCHUNK - BASE_CHUNKS * NW
NP = 10240
RPT = NP // NS

R = 1000
GRID = N // R


def _mesh():
    return plsc.VectorSubcoreMesh(
        core_axis_name="c", subcore_axis_name="s", num_cores=NC, num_subcores=NS
    )


EP = 327680
CPW = EP // C // NW
IB = 40
GROUPS = CPW // IB


@functools.lru_cache(maxsize=None)
def _seg_sum_kernel():
    @functools.partial(
        pl.kernel,
        out_type=jax.ShapeDtypeStruct((2 * NP, H), jnp.float32),
        mesh=_mesh(),
        scratch_types=[
            pltpu.VMEM((IB, C), jnp.int32),
            pltpu.VMEM((IB, C), jnp.int32),
            pltpu.VMEM((C, H), jnp.float32),
            pltpu.VMEM((C, H), jnp.float32),
            pltpu.VMEM_SHARED((NP, H), jnp.float32),
            pltpu.SemaphoreType.DMA,
            pltpu.SemaphoreType.DMA,
            pltpu.SemaphoreType.DMA,
            pltpu.SemaphoreType.DMA,
        ],
        name="sc_seg_sum",
    )
    def k(src_hbm, dst_hbm, h_hbm, zeros_hbm, out_hbm,
          srcb_v, dstb_v, rows0_v, rows1_v,
          part_s, sem_g0, sem_g1, sem_s0, sem_s1):
        cid = lax.axis_index("c")
        sid = lax.axis_index("s")
        w = sid * NC + cid
        chunk0 = w * CPW

        pltpu.sync_copy(zeros_hbm.at[pl.ds(sid * RPT, RPT)],
                        part_s.at[pl.ds(sid * RPT, RPT)])
        plsc.subcore_barrier()

        rows = (rows0_v, rows1_v)
        sem_g = (sem_g0, sem_g1)
        sem_s = (sem_s0, sem_s1)

        ngroups = (CPW // IB) + (w < 0).astype(jnp.int32)

        def body(t, carry):
            c0 = chunk0 + t * IB
            pltpu.sync_copy(src_hbm.at[pl.ds(c0, IB)], srcb_v)
            pltpu.sync_copy(dst_hbm.at[pl.ds(c0, IB)], dstb_v)
            g = [None] * IB
            sct = [None] * IB
            g[0] = pltpu.async_copy(h_hbm.at[srcb_v.at[0]], rows[0], sem_g[0])
            g[1] = pltpu.async_copy(h_hbm.at[srcb_v.at[1]], rows[1], sem_g[1])
            for j in range(IB):
                p = j % 2
                g[j].wait()
                sct[j] = pltpu.async_copy(rows[p], part_s.at[dstb_v.at[j]],
                                          sem_s[p], add=True)
                if j + 2 < IB:
                    sct[j].wait()
                    g[j + 2] = pltpu.async_copy(h_hbm.at[srcb_v.at[j + 2]],
                                                rows[p], sem_g[p])
            sct[IB - 2].wait()
            sct[IB - 1].wait()
            return carry

        lax.fori_loop(0, ngroups, body, 0)
        plsc.subcore_barrier()
        row0 = cid * NP + sid * RPT
        pltpu.sync_copy(part_s.at[pl.ds(sid * RPT, RPT)],
                        out_hbm.at[pl.ds(row0, RPT)])

    return k


@functools.lru_cache(maxsize=None)
def _feas_kernel():
    @functools.partial(
        pl.kernel,
        out_type=jax.ShapeDtypeStruct((NW * 16,), jnp.float32),
        mesh=_mesh(),
        scratch_types=[
            pltpu.VMEM((C,), jnp.int32),
            pltpu.VMEM((C,), jnp.int32),
            pltpu.VMEM((C,), jnp.float32),
            pltpu.VMEM((C,), jnp.float32),
            pltpu.VMEM((16,), jnp.float32),
            pltpu.SemaphoreType.DMA,
        ],
        name="sc_feas",
    )
    def k(probs_hbm, src_hbm, dst_hbm, out_hbm,
          si_v, di_v, sv_v, dv_v, acc_v, sem):
        cid = lax.axis_index("c")
        sid = lax.axis_index("s")
        w = sid * NC + cid
        nchunks = BASE_CHUNKS + (w < EXTRA).astype(jnp.int32)

        def body(i, acc):
            base = (w + i * NW) * C
            pltpu.sync_copy(src_hbm.at[pl.ds(base, C)], si_v)
            pltpu.sync_copy(dst_hbm.at[pl.ds(base, C)], di_v)
            pltpu.async_copy(probs_hbm.at[si_v], sv_v, sem).wait()
            pltpu.async_copy(probs_hbm.at[di_v], dv_v, sem).wait()
            for j in range(C // 16):
                acc = acc + sv_v[pl.ds(j * 16, 16)] * dv_v[pl.ds(j * 16, 16)]
            return acc

        acc = lax.fori_loop(0, nchunks, body, jnp.zeros((16,), jnp.float32))
        acc_v[...] = acc
        pltpu.sync_copy(acc_v, out_hbm.at[pl.ds(w * 16, 16)])

    return k


def _ln(t, g, b, eps=1e-5):
    m = jnp.mean(t, axis=-1, keepdims=True)
    tc = t - m
    v = jnp.mean(tc * tc, axis=-1, keepdims=True)
    return tc * lax.rsqrt(v + eps) * g + b


def _full(shape):
    return pl.BlockSpec(shape, lambda i: (0,) * len(shape))


def _rows(width):
    return pl.BlockSpec((R, width), lambda i: (i, 0))


def _dot(a, b):
    return jnp.dot(a, b, preferred_element_type=jnp.float32)


def _embed_body(x_ref, xw_ref, xb_ref, g_ref, b_ref, wpx_ref, c0_ref):
    xx = x_ref[...]
    e = xx[:, 0:1] * xw_ref[0:1, :] + xx[:, 1:2] * xw_ref[1:2, :] + xb_ref[...]
    e = _ln(e, g_ref[...], b_ref[...])
    c0_ref[...] = _dot(e, wpx_ref[...])


def _embed(x, xw, xb, g, b, wpx):
    return pl.pallas_call(
        _embed_body,
        grid=(GRID,),
        in_specs=[_rows(2), _full((2, H)), _full((1, H)), _full((1, H)),
                  _full((1, H)), _full((H, H))],
        out_specs=_rows(H),
        out_shape=jax.ShapeDtypeStruct((N, H), jnp.float32),
    )(x, xw, xb, g, b, wpx)


def _stepin_body(c0_ref, y_ref, z_ref, wpy_ref, wpz_ref, bp_ref, g_ref, b_ref,
                 h_ref):
    t = (c0_ref[...] + jax.nn.sigmoid(y_ref[...]) * wpy_ref[...]
         + _dot(z_ref[...], wpz_ref[...]) + bp_ref[...])
    h_ref[...] = _ln(t, g_ref[...], b_ref[...])


def _stepin(c0, y, z, wpy, wpz, bp, g, b):
    return pl.pallas_call(
        _stepin_body,
        grid=(GRID,),
        in_specs=[_rows(H), _rows(1), _rows(H), _full((1, H)), _full((H, H)),
                  _full((1, H)), _full((1, H)), _full((1, H))],
        out_specs=_rows(H),
        out_shape=jax.ShapeDtypeStruct((N, H), jnp.float32),
    )(c0, y, z, wpy, wpz, bp, g, b)


def _gin_body(h_ref, p0_ref, p1_ref, eps_ref, w1_ref, b1_ref, g1_ref, bb1_ref,
              w2_ref, b2_ref, pg_ref, pb_ref, out_ref):
    h = h_ref[...]
    u = (1.0 + eps_ref[0, 0]) * h + p0_ref[...] + p1_ref[...]
    t = _dot(u, w1_ref[...]) + b1_ref[...]
    t = jax.nn.gelu(_ln(t, g1_ref[...], bb1_ref[...]))
    v = _dot(t, w2_ref[...]) + b2_ref[...]
    out_ref[...] = _ln(h + jax.nn.gelu(v), pg_ref[...], pb_ref[...])


def _gin_post(h, p0, p1, eps, w1, b1, g1, bb1, w2, b2, pg, pb):
    return pl.pallas_call(
        _gin_body,
        grid=(GRID,),
        in_specs=[_rows(H), _rows(H), _rows(H),
                  pl.BlockSpec(memory_space=pltpu.SMEM),
                  _full((H, 2 * H)), _full((1, 2 * H)), _full((1, 2 * H)),
                  _full((1, 2 * H)), _full((2 * H, H)), _full((1, H)),
                  _full((1, H)), _full((1, H))],
        out_specs=_rows(H),
        out_shape=jax.ShapeDtypeStruct((N, H), jnp.float32),
    )(h, p0, p1, eps, w1, b1, g1, bb1, w2, b2, pg, pb)


def _outstep_body(y_ref, z_ref, woy_ref, woz_ref, bo_ref, og_ref, ob_ref,
                  w1_ref, b1_ref, w2_ref, b2_ref, yo_ref):
    t = (y_ref[...] * woy_ref[...] + _dot(z_ref[...], woz_ref[...])
         + bo_ref[...])
    t = _ln(t, og_ref[...], ob_ref[...])
    g = jax.nn.gelu(_dot(t, w1_ref[...]) + b1_ref[...])
    yo_ref[...] = _dot(g, w2_ref[...]) + b2_ref[0, 0]


def _outstep(y, z, woy, woz, bo, og, ob, w1, b1, w2, b2):
    return pl.pallas_call(
        _outstep_body,
        grid=(GRID,),
        in_specs=[_rows(1), _rows(H), _full((1, H)), _full((H, H)),
                  _full((1, H)), _full((1, H)), _full((1, H)),
                  _full((H, H)), _full((1, H)), _full((H, 1)),
                  pl.BlockSpec(memory_space=pltpu.SMEM)],
        out_specs=_rows(1),
        out_shape=jax.ShapeDtypeStruct((N, 1), jnp.float32),
    )(y, z, woy, woz, bo, og, ob, w1, b1, w2, b2)


def _probs_body(y_ref, p_ref):
    p_ref[...] = jax.nn.sigmoid(jnp.clip(y_ref[...], -10.0, 10.0))


def _probs(y):
    return pl.pallas_call(
        _probs_body,
        grid=(GRID,),
        in_specs=[_rows(1)],
        out_specs=_rows(1),
        out_shape=jax.ShapeDtypeStruct((N, 1), jnp.float32),
    )(y)


def _loss_body(y_ref, lab_ref, fp_ref, a_ref, b_ref, p_ref, f_ref):
    i = pl.program_id(0)
    l = jnp.clip(y_ref[...], -10.0, 10.0)
    lab = lab_ref[...].astype(jnp.float32)
    a = jnp.sum(lab * jax.nn.softplus(-l))
    b = jnp.sum((1.0 - lab) * jax.nn.softplus(l))
    p = jnp.sum(lab)

    @pl.when(i == 0)
    def _():
        zz = jnp.zeros((1, 1), jnp.float32)
        a_ref[...] = zz
        b_ref[...] = zz
        p_ref[...] = zz
        f_ref[...] = jnp.sum(fp_ref[...]).reshape(1, 1)

    a_ref[...] = a_ref[...] + a
    b_ref[...] = b_ref[...] + b
    p_ref[...] = p_ref[...] + p


def _loss(y, labels2d, feas_parts):
    s = jax.ShapeDtypeStruct((1, 1), jnp.float32)
    one = pl.BlockSpec((1, 1), lambda i: (0, 0))
    return pl.pallas_call(
        _loss_body,
        grid=(GRID,),
        in_specs=[_rows(1), _rows(1), pl.BlockSpec((1, NW * 16), lambda i: (0, 0))],
        out_specs=(one, one, one, one),
        out_shape=(s, s, s, s),
    )(y, labels2d, feas_parts)


def _gin2s_body(h_ref, p0_ref, p1_ref, eps_ref, w1_ref, b1_ref, g1_ref,
                bb1_ref, w2_ref, b2_ref, pg_ref, pb_ref,
                c0_ref, y_ref, wpy_ref, wpz_ref, bp_ref, lg_ref, lb_ref,
                out_ref):
    h = h_ref[...]
    u = (1.0 + eps_ref[0, 0]) * h + p0_ref[...] + p1_ref[...]
    t = _dot(u, w1_ref[...]) + b1_ref[...]
    t = jax.nn.gelu(_ln(t, g1_ref[...], bb1_ref[...]))
    v = _dot(t, w2_ref[...]) + b2_ref[...]
    h2 = _ln(h + jax.nn.gelu(v), pg_ref[...], pb_ref[...])
    tt = (c0_ref[...] + jax.nn.sigmoid(y_ref[...]) * wpy_ref[...]
          + _dot(h2, wpz_ref[...]) + bp_ref[...])
    out_ref[...] = _ln(tt, lg_ref[...], lb_ref[...])


def _gin2s(h, p0, p1, eps, w1, b1, g1, bb1, w2, b2, pg, pb,
           c0, y, wpy, wpz, bp, lg, lb):
    return pl.pallas_call(
        _gin2s_body,
        grid=(GRID,),
        in_specs=[_rows(H), _rows(H), _rows(H),
                  pl.BlockSpec(memory_space=pltpu.SMEM),
                  _full((H, 2 * H)), _full((1, 2 * H)), _full((1, 2 * H)),
                  _full((1, 2 * H)), _full((2 * H, H)), _full((1, H)),
                  _full((1, H)), _full((1, H)),
                  _rows(H), _rows(1), _full((1, H)), _full((H, H)),
                  _full((1, H)), _full((1, H)), _full((1, H))],
        out_specs=_rows(H),
        out_shape=jax.ShapeDtypeStruct((N, H), jnp.float32),
    )(h, p0, p1, eps, w1, b1, g1, bb1, w2, b2, pg, pb,
      c0, y, wpy, wpz, bp, lg, lb)


def kernel(x, edge_index, y_carry, z_carry, labels, H_step, params):
    p = params
    src = edge_index[0]
    dst = edge_index[1]
    zeros = jnp.zeros((NP, H), jnp.float32)
    pad = EP - E
    pad_dst = N + (jnp.arange(pad, dtype=jnp.int32) % (NP - N))
    pad_src = jnp.arange(pad, dtype=jnp.int32) % N
    srcp = jnp.concatenate([src, pad_src]).reshape(-1, C)
    dstp = jnp.concatenate([dst, pad_dst]).reshape(-1, C)

    wp = p["latent_proj_w"]
    wpx, wpy, wpz = wp[:H], wp[H:H + 1], wp[H + 1:]
    bp = p["latent_proj_b"].reshape(1, H)
    lng, lnb = p["latent_norm_g"].reshape(1, H), p["latent_norm_b"].reshape(1, H)

    wo = p["output_proj_w"]
    woy, woz = wo[:1], wo[1:]
    bo = p["output_norm_b"]

    c0 = _embed(x, p["x_embed_w"], p["x_embed_b"].reshape(1, H),
                p["x_norm_g"].reshape(1, H), p["x_norm_b"].reshape(1, H), wpx)

    seg = _seg_sum_kernel()
    feask = _feas_kernel()

    gins = []
    for gp in p["gin"]:
        gins.append((
            gp["eps"].reshape(1, 1),
            gp["w1"], gp["b1"].reshape(1, 2 * H),
            gp["ln_g"].reshape(1, 2 * H), gp["ln_b"].reshape(1, 2 * H),
            gp["w2"], gp["b2"].reshape(1, H),
            gp["post_ln_g"].reshape(1, H), gp["post_ln_b"].reshape(1, H),
        ))

    y, z = y_carry, z_carry
    L_CYCLES, H_CYCLES = 6, 3
    g0, g1 = gins
    for _ in range(H_CYCLES):
        h = _stepin(c0, y, z, wpy, wpz, bp, lng, lnb)
        for l in range(L_CYCLES):
            parts = seg(srcp, dstp, h, zeros)
            h = _gin_post(h, parts[:N], parts[NP:NP + N], *g0)
            parts = seg(srcp, dstp, h, zeros)
            if l < L_CYCLES - 1:
                h = _gin2s(h, parts[:N], parts[NP:NP + N], *g1,
                           c0, y, wpy, wpz, bp, lng, lnb)
            else:
                z = _gin_post(h, parts[:N], parts[NP:NP + N], *g1)
        y = _outstep(y, z, woy, woz, p["output_proj_b"].reshape(1, H),
                     p["output_norm_g"].reshape(1, H),
                     p["output_norm_b"].reshape(1, H),
                     p["head_w1"], p["head_b1"].reshape(1, H),
                     p["head_w2"], p["head_b2"].reshape(1, 1))

    probs = _probs(y)
    feas_parts = feask(probs.reshape(-1), src, dst)
    a, b, pcnt, fsum = _loss(y, labels.reshape(N, 1), feas_parts.reshape(1, NW * 16))

    pos = jnp.clip(pcnt[0, 0], 1.0, None)
    neg = jnp.clip(float(N) - pos, 1.0, None)
    pw = neg / pos
    bce = (pw * a[0, 0] + b[0, 0]) / float(N)
    feas = fsum[0, 0] / float(E)
    return bce + 50.0 * feas


# speedup vs baseline: 3.8894x; 1.0210x over previous
import functools

import jax
import jax.numpy as jnp
from jax import lax
from jax.experimental import pallas as pl
from jax.experimental.pallas import tpu as pltpu
from jax.experimental.pallas import tpu_sc as plsc

N = 10000
E = 320000
H = 128
NC = 2
NS = 16
NW = NC * NS
C = 128
NCHUNK = E // C
BASE_CHUNKS = NCHUNK // NW
EXTRA = NCHUNK - BASE_CHUNKS * NW
NP = 10240
RPT = NP // NS

R = 1000
GRID = N // R


def _mesh():
    return plsc.VectorSubcoreMesh(
        core_axis_name="c", subcore_axis_name="s", num_cores=NC, num_subcores=NS
    )


EP = 327680
CPW = EP // C // NW
IB = 40
GROUPS = CPW // IB


@functools.lru_cache(maxsize=None)
def _seg_sum_kernel():
    @functools.partial(
        pl.kernel,
        out_type=jax.ShapeDtypeStruct((2 * NP, H), jnp.float32),
        mesh=_mesh(),
        scratch_types=[
            pltpu.VMEM((IB, C), jnp.int32),
            pltpu.VMEM((IB, C), jnp.int32),
            pltpu.VMEM((C, H), jnp.float32),
            pltpu.VMEM((C, H), jnp.float32),
            pltpu.VMEM_SHARED((NP, H), jnp.float32),
            pltpu.SemaphoreType.DMA,
            pltpu.SemaphoreType.DMA,
            pltpu.SemaphoreType.DMA,
            pltpu.SemaphoreType.DMA,
        ],
        name="sc_seg_sum",
    )
    def k(src_hbm, dst_hbm, h_hbm, zeros_hbm, out_hbm,
          srcb_v, dstb_v, rows0_v, rows1_v,
          part_s, sem_g0, sem_g1, sem_s0, sem_s1):
        cid = lax.axis_index("c")
        sid = lax.axis_index("s")
        w = sid * NC + cid
        chunk0 = w * CPW

        pltpu.sync_copy(zeros_hbm.at[pl.ds(sid * RPT, RPT)],
                        part_s.at[pl.ds(sid * RPT, RPT)])
        plsc.subcore_barrier()

        rows = (rows0_v, rows1_v)
        sem_g = (sem_g0, sem_g1)
        sem_s = (sem_s0, sem_s1)

        ngroups = (CPW // IB) + (w < 0).astype(jnp.int32)

        def body(t, carry):
            c0 = chunk0 + t * IB
            pltpu.sync_copy(src_hbm.at[pl.ds(c0, IB)], srcb_v)
            pltpu.sync_copy(dst_hbm.at[pl.ds(c0, IB)], dstb_v)
            g = [None] * IB
            sct = [None] * IB
            g[0] = pltpu.async_copy(h_hbm.at[srcb_v.at[0]], rows[0], sem_g[0])
            g[1] = pltpu.async_copy(h_hbm.at[srcb_v.at[1]], rows[1], sem_g[1])
            for j in range(IB):
                p = j % 2
                g[j].wait()
                sct[j] = pltpu.async_copy(rows[p], part_s.at[dstb_v.at[j]],
                                          sem_s[p], add=True)
                if j + 2 < IB:
                    sct[j].wait()
                    g[j + 2] = pltpu.async_copy(h_hbm.at[srcb_v.at[j + 2]],
                                                rows[p], sem_g[p])
            sct[IB - 2].wait()
            sct[IB - 1].wait()
            return carry

        lax.fori_loop(0, ngroups, body, 0)
        plsc.subcore_barrier()
        row0 = cid * NP + sid * RPT
        pltpu.sync_copy(part_s.at[pl.ds(sid * RPT, RPT)],
                        out_hbm.at[pl.ds(row0, RPT)])

    return k


@functools.lru_cache(maxsize=None)
def _feas_kernel():
    @functools.partial(
        pl.kernel,
        out_type=jax.ShapeDtypeStruct((NW * 16,), jnp.float32),
        mesh=_mesh(),
        scratch_types=[
            pltpu.VMEM((IB, C), jnp.int32),
            pltpu.VMEM((IB, C), jnp.int32),
            pltpu.VMEM((C,), jnp.float32),
            pltpu.VMEM((C,), jnp.float32),
            pltpu.VMEM((16,), jnp.float32),
            pltpu.SemaphoreType.DMA,
            pltpu.SemaphoreType.DMA,
        ],
        name="sc_feas",
    )
    def k(probs_hbm, src_hbm, dst_hbm, out_hbm,
          srcb_v, dstb_v, sv_v, dv_v, acc_v, sem_a, sem_b):
        cid = lax.axis_index("c")
        sid = lax.axis_index("s")
        w = sid * NC + cid
        chunk0 = w * CPW
        ngroups = (CPW // IB) + (w < 0).astype(jnp.int32)

        def body(t, acc):
            pltpu.sync_copy(src_hbm.at[pl.ds(chunk0 + t * IB, IB)], srcb_v)
            pltpu.sync_copy(dst_hbm.at[pl.ds(chunk0 + t * IB, IB)], dstb_v)
            for j in range(IB):
                ga = pltpu.async_copy(probs_hbm.at[srcb_v.at[j]], sv_v, sem_a)
                gb = pltpu.async_copy(probs_hbm.at[dstb_v.at[j]], dv_v, sem_b)
                ga.wait()
                gb.wait()
                for q in range(C // 16):
                    acc = acc + (sv_v[pl.ds(q * 16, 16)]
                                 * dv_v[pl.ds(q * 16, 16)])
            return acc

        acc = lax.fori_loop(0, ngroups, body, jnp.zeros((16,), jnp.float32))
        acc_v[...] = acc
        pltpu.sync_copy(acc_v, out_hbm.at[pl.ds(w * 16, 16)])

    return k


def _ln(t, g, b, eps=1e-5):
    m = jnp.mean(t, axis=-1, keepdims=True)
    tc = t - m
    v = jnp.mean(tc * tc, axis=-1, keepdims=True)
    return tc * lax.rsqrt(v + eps) * g + b


def _full(shape):
    return pl.BlockSpec(shape, lambda i: (0,) * len(shape))


def _rows(width):
    return pl.BlockSpec((R, width), lambda i: (i, 0))


def _dot(a, b):
    return jnp.dot(a, b, preferred_element_type=jnp.float32)


def _embed_body(x_ref, xw_ref, xb_ref, g_ref, b_ref, wpx_ref, c0_ref):
    xx = x_ref[...]
    e = xx[:, 0:1] * xw_ref[0:1, :] + xx[:, 1:2] * xw_ref[1:2, :] + xb_ref[...]
    e = _ln(e, g_ref[...], b_ref[...])
    c0_ref[...] = _dot(e, wpx_ref[...])


def _embed(x, xw, xb, g, b, wpx):
    return pl.pallas_call(
        _embed_body,
        grid=(GRID,),
        in_specs=[_rows(2), _full((2, H)), _full((1, H)), _full((1, H)),
                  _full((1, H)), _full((H, H))],
        out_specs=_rows(H),
        out_shape=jax.ShapeDtypeStruct((N, H), jnp.float32),
    )(x, xw, xb, g, b, wpx)


def _stepin_body(c0_ref, y_ref, z_ref, wpy_ref, wpz_ref, bp_ref, g_ref, b_ref,
                 h_ref):
    t = (c0_ref[...] + jax.nn.sigmoid(y_ref[...]) * wpy_ref[...]
         + _dot(z_ref[...], wpz_ref[...]) + bp_ref[...])
    h_ref[...] = _ln(t, g_ref[...], b_ref[...])


def _stepin(c0, y, z, wpy, wpz, bp, g, b):
    return pl.pallas_call(
        _stepin_body,
        grid=(GRID,),
        in_specs=[_rows(H), _rows(1), _rows(H), _full((1, H)), _full((H, H)),
                  _full((1, H)), _full((1, H)), _full((1, H))],
        out_specs=_rows(H),
        out_shape=jax.ShapeDtypeStruct((N, H), jnp.float32),
    )(c0, y, z, wpy, wpz, bp, g, b)


def _gin_body(h_ref, p0_ref, p1_ref, eps_ref, w1_ref, b1_ref, g1_ref, bb1_ref,
              w2_ref, b2_ref, pg_ref, pb_ref, out_ref):
    h = h_ref[...]
    u = (1.0 + eps_ref[0, 0]) * h + p0_ref[...] + p1_ref[...]
    t = _dot(u, w1_ref[...]) + b1_ref[...]
    t = jax.nn.gelu(_ln(t, g1_ref[...], bb1_ref[...]))
    v = _dot(t, w2_ref[...]) + b2_ref[...]
    out_ref[...] = _ln(h + jax.nn.gelu(v), pg_ref[...], pb_ref[...])


def _gin_post(h, p0, p1, eps, w1, b1, g1, bb1, w2, b2, pg, pb):
    return pl.pallas_call(
        _gin_body,
        grid=(GRID,),
        in_specs=[_rows(H), _rows(H), _rows(H),
                  pl.BlockSpec(memory_space=pltpu.SMEM),
                  _full((H, 2 * H)), _full((1, 2 * H)), _full((1, 2 * H)),
                  _full((1, 2 * H)), _full((2 * H, H)), _full((1, H)),
                  _full((1, H)), _full((1, H))],
        out_specs=_rows(H),
        out_shape=jax.ShapeDtypeStruct((N, H), jnp.float32),
    )(h, p0, p1, eps, w1, b1, g1, bb1, w2, b2, pg, pb)


def _outstep_body(y_ref, z_ref, woy_ref, woz_ref, bo_ref, og_ref, ob_ref,
                  w1_ref, b1_ref, w2_ref, b2_ref, yo_ref):
    t = (y_ref[...] * woy_ref[...] + _dot(z_ref[...], woz_ref[...])
         + bo_ref[...])
    t = _ln(t, og_ref[...], ob_ref[...])
    g = jax.nn.gelu(_dot(t, w1_ref[...]) + b1_ref[...])
    yo_ref[...] = _dot(g, w2_ref[...]) + b2_ref[0, 0]


def _outstep(y, z, woy, woz, bo, og, ob, w1, b1, w2, b2):
    return pl.pallas_call(
        _outstep_body,
        grid=(GRID,),
        in_specs=[_rows(1), _rows(H), _full((1, H)), _full((H, H)),
                  _full((1, H)), _full((1, H)), _full((1, H)),
                  _full((H, H)), _full((1, H)), _full((H, 1)),
                  pl.BlockSpec(memory_space=pltpu.SMEM)],
        out_specs=_rows(1),
        out_shape=jax.ShapeDtypeStruct((N, 1), jnp.float32),
    )(y, z, woy, woz, bo, og, ob, w1, b1, w2, b2)


def _probs_body(y_ref, p_ref):
    p_ref[...] = jax.nn.sigmoid(jnp.clip(y_ref[...], -10.0, 10.0))


def _probs(y):
    return pl.pallas_call(
        _probs_body,
        grid=(GRID,),
        in_specs=[_rows(1)],
        out_specs=_rows(1),
        out_shape=jax.ShapeDtypeStruct((N, 1), jnp.float32),
    )(y)


def _loss_body(y_ref, lab_ref, fp_ref, a_ref, b_ref, p_ref, f_ref):
    i = pl.program_id(0)
    l = jnp.clip(y_ref[...], -10.0, 10.0)
    lab = lab_ref[...].astype(jnp.float32)
    a = jnp.sum(lab * jax.nn.softplus(-l))
    b = jnp.sum((1.0 - lab) * jax.nn.softplus(l))
    p = jnp.sum(lab)

    @pl.when(i == 0)
    def _():
        zz = jnp.zeros((1, 1), jnp.float32)
        a_ref[...] = zz
        b_ref[...] = zz
        p_ref[...] = zz
        f_ref[...] = jnp.sum(fp_ref[...]).reshape(1, 1)

    a_ref[...] = a_ref[...] + a
    b_ref[...] = b_ref[...] + b
    p_ref[...] = p_ref[...] + p


def _loss(y, labels2d, feas_parts):
    s = jax.ShapeDtypeStruct((1, 1), jnp.float32)
    one = pl.BlockSpec((1, 1), lambda i: (0, 0))
    return pl.pallas_call(
        _loss_body,
        grid=(GRID,),
        in_specs=[_rows(1), _rows(1), pl.BlockSpec((1, NW * 16), lambda i: (0, 0))],
        out_specs=(one, one, one, one),
        out_shape=(s, s, s, s),
    )(y, labels2d, feas_parts)


def _gin2s_body(h_ref, p0_ref, p1_ref, eps_ref, w1_ref, b1_ref, g1_ref,
                bb1_ref, w2_ref, b2_ref, pg_ref, pb_ref,
                c0_ref, y_ref, wpy_ref, wpz_ref, bp_ref, lg_ref, lb_ref,
                out_ref):
    h = h_ref[...]
    u = (1.0 + eps_ref[0, 0]) * h + p0_ref[...] + p1_ref[...]
    t = _dot(u, w1_ref[...]) + b1_ref[...]
    t = jax.nn.gelu(_ln(t, g1_ref[...], bb1_ref[...]))
    v = _dot(t, w2_ref[...]) + b2_ref[...]
    h2 = _ln(h + jax.nn.gelu(v), pg_ref[...], pb_ref[...])
    tt = (c0_ref[...] + jax.nn.sigmoid(y_ref[...]) * wpy_ref[...]
          + _dot(h2, wpz_ref[...]) + bp_ref[...])
    out_ref[...] = _ln(tt, lg_ref[...], lb_ref[...])


def _gin2s(h, p0, p1, eps, w1, b1, g1, bb1, w2, b2, pg, pb,
           c0, y, wpy, wpz, bp, lg, lb):
    return pl.pallas_call(
        _gin2s_body,
        grid=(GRID,),
        in_specs=[_rows(H), _rows(H), _rows(H),
                  pl.BlockSpec(memory_space=pltpu.SMEM),
                  _full((H, 2 * H)), _full((1, 2 * H)), _full((1, 2 * H)),
                  _full((1, 2 * H)), _full((2 * H, H)), _full((1, H)),
                  _full((1, H)), _full((1, H)),
                  _rows(H), _rows(1), _full((1, H)), _full((H, H)),
                  _full((1, H)), _full((1, H)), _full((1, H))],
        out_specs=_rows(H),
        out_shape=jax.ShapeDtypeStruct((N, H), jnp.float32),
    )(h, p0, p1, eps, w1, b1, g1, bb1, w2, b2, pg, pb,
      c0, y, wpy, wpz, bp, lg, lb)


def kernel(x, edge_index, y_carry, z_carry, labels, H_step, params):
    p = params
    src = edge_index[0]
    dst = edge_index[1]
    zeros = jnp.zeros((NP, H), jnp.float32)
    pad = EP - E
    pad_dst = N + (jnp.arange(pad, dtype=jnp.int32) % (NP - N))
    pad_src = jnp.arange(pad, dtype=jnp.int32) % N
    srcp = jnp.concatenate([src, pad_src]).reshape(-1, C)
    dstp = jnp.concatenate([dst, pad_dst]).reshape(-1, C)

    wp = p["latent_proj_w"]
    wpx, wpy, wpz = wp[:H], wp[H:H + 1], wp[H + 1:]
    bp = p["latent_proj_b"].reshape(1, H)
    lng, lnb = p["latent_norm_g"].reshape(1, H), p["latent_norm_b"].reshape(1, H)

    wo = p["output_proj_w"]
    woy, woz = wo[:1], wo[1:]
    bo = p["output_norm_b"]

    c0 = _embed(x, p["x_embed_w"], p["x_embed_b"].reshape(1, H),
                p["x_norm_g"].reshape(1, H), p["x_norm_b"].reshape(1, H), wpx)

    seg = _seg_sum_kernel()
    feask = _feas_kernel()

    gins = []
    for gp in p["gin"]:
        gins.append((
            gp["eps"].reshape(1, 1),
            gp["w1"], gp["b1"].reshape(1, 2 * H),
            gp["ln_g"].reshape(1, 2 * H), gp["ln_b"].reshape(1, 2 * H),
            gp["w2"], gp["b2"].reshape(1, H),
            gp["post_ln_g"].reshape(1, H), gp["post_ln_b"].reshape(1, H),
        ))

    y, z = y_carry, z_carry
    L_CYCLES, H_CYCLES = 6, 3
    g0, g1 = gins
    for _ in range(H_CYCLES):
        h = _stepin(c0, y, z, wpy, wpz, bp, lng, lnb)
        for l in range(L_CYCLES):
            parts = seg(srcp, dstp, h, zeros)
            h = _gin_post(h, parts[:N], parts[NP:NP + N], *g0)
            parts = seg(srcp, dstp, h, zeros)
            if l < L_CYCLES - 1:
                h = _gin2s(h, parts[:N], parts[NP:NP + N], *g1,
                           c0, y, wpy, wpz, bp, lng, lnb)
            else:
                z = _gin_post(h, parts[:N], parts[NP:NP + N], *g1)
        y = _outstep(y, z, woy, woz, p["output_proj_b"].reshape(1, H),
                     p["output_norm_g"].reshape(1, H),
                     p["output_norm_b"].reshape(1, H),
                     p["head_w1"], p["head_b1"].reshape(1, H),
                     p["head_w2"], p["head_b2"].reshape(1, 1))

    probs = _probs(y)
    probs_ext = jnp.concatenate([probs.reshape(-1),
                                 jnp.zeros((NP - N,), jnp.float32)])
    feas_parts = feask(probs_ext, srcp, dstp)
    a, b, pcnt, fsum = _loss(y, labels.reshape(N, 1), feas_parts.reshape(1, NW * 16))

    pos = jnp.clip(pcnt[0, 0], 1.0, None)
    neg = jnp.clip(float(N) - pos, 1.0, None)
    pw = neg / pos
    bce = (pw * a[0, 0] + b[0, 0]) / float(N)
    feas = fsum[0, 0] / float(E)
    return bce + 50.0 * feas
